# Initial kernel scaffold; baseline (speedup 1.0000x reference)
import jax
import jax.numpy as jnp
from jax.experimental import pallas as pl


def kernel(x, edge_index, batch, W_lin1, b_lin1, W_lin2, b_lin2, W_conv1, b_conv1, W_conv2, b_conv2, W_conv3, b_conv3, att1, att2, W_fc1, b_fc1, W_fc2, b_fc2, W_fc3, b_fc3):
    raise NotImplementedError("write your pallas kernel here")


# speedup vs baseline: 1.0131x; 1.0131x over previous
import functools
from functools import partial

import jax
import jax.numpy as jnp
import numpy as np
from jax.experimental import pallas as pl
from jax.experimental.pallas import tpu as pltpu

N0 = 10000
E = 320000
AA = 21
H = 128
LAMB = 1.0
K1 = 5000
K2 = 2500


def _front_body(x_ref, w2p_ref, b2p_ref, wc1a_ref, w1p_ref, b1p_ref,
                wc1b_ref, hm_ref):
    x = x_ref[...]
    xb = jnp.maximum(jnp.dot(x, w2p_ref[...],
                             preferred_element_type=jnp.float32)
                     + b2p_ref[...], 0.0)
    xa = jnp.maximum(jnp.dot(x, w1p_ref[...],
                             preferred_element_type=jnp.float32)
                     + b1p_ref[...], 0.0)
    hm_ref[...] = (jnp.dot(xb, wc1a_ref[...], preferred_element_type=jnp.float32)
                   + jnp.dot(xa, wc1b_ref[...], preferred_element_type=jnp.float32))


def _front(x, W_lin1, b_lin1, W_lin2, b_lin2, W_conv1, b_conv1):
    W2p = jnp.zeros((H, H), jnp.float32).at[:AA, :AA].set(W_lin2)
    b2p = jnp.zeros((1, H), jnp.float32).at[0, :AA].set(b_lin2)
    Wc1a = jnp.zeros((H, H), jnp.float32).at[:AA, :].set(W_conv1[:AA])
    W1p = jnp.zeros((H, H), jnp.float32).at[AA:, :].set(W_lin1)
    b1p = jnp.broadcast_to(b_lin1[None, :], (1, H))
    Wc1b = W_conv1[AA:]
    n = x.shape[0]
    bm = 512
    grid = (pl.cdiv(n, bm),)
    return pl.pallas_call(
        _front_body,
        grid=grid,
        in_specs=[pl.BlockSpec((bm, H), lambda i: (i, 0))] +
                 [pl.BlockSpec((H, H), lambda i: (0, 0)),
                  pl.BlockSpec((1, H), lambda i: (0, 0)),
                  pl.BlockSpec((H, H), lambda i: (0, 0)),
                  pl.BlockSpec((H, H), lambda i: (0, 0)),
                  pl.BlockSpec((1, H), lambda i: (0, 0)),
                  pl.BlockSpec((H, H), lambda i: (0, 0))],
        out_specs=pl.BlockSpec((bm, H), lambda i: (i, 0)),
        out_shape=jax.ShapeDtypeStruct((n, H), jnp.float32),
    )(x, W2p, b2p, Wc1a, W1p, b1p, Wc1b)


def _matmul_body(x_ref, w_ref, o_ref):
    o_ref[...] = jnp.dot(x_ref[...], w_ref[...],
                         preferred_element_type=jnp.float32)


def _matmul(x, w):
    n = x.shape[0]
    bm = 512
    return pl.pallas_call(
        _matmul_body,
        grid=(pl.cdiv(n, bm),),
        in_specs=[pl.BlockSpec((bm, H), lambda i: (i, 0)),
                  pl.BlockSpec((H, H), lambda i: (0, 0))],
        out_specs=pl.BlockSpec((bm, H), lambda i: (i, 0)),
        out_shape=jax.ShapeDtypeStruct((n, H), jnp.float32),
    )(x, w)


def _gcn_combine_body(a0_ref, a1_ref, hm_ref, dinv_ref, b_ref, o_ref):
    dinv = dinv_ref[...]
    o_ref[...] = jnp.maximum(
        a0_ref[...] + a1_ref[...] + dinv * dinv * hm_ref[...] + b_ref[...], 0.0)


def _gcn_combine(a0, a1, hm, dinv_col, b):
    n = hm.shape[0]
    bm = 512
    return pl.pallas_call(
        _gcn_combine_body,
        grid=(pl.cdiv(n, bm),),
        in_specs=[pl.BlockSpec((bm, H), lambda i: (i, 0)),
                  pl.BlockSpec((bm, H), lambda i: (i, 0)),
                  pl.BlockSpec((bm, H), lambda i: (i, 0)),
                  pl.BlockSpec((bm, 1), lambda i: (i, 0)),
                  pl.BlockSpec((1, H), lambda i: (0, 0))],
        out_specs=pl.BlockSpec((bm, H), lambda i: (i, 0)),
        out_shape=jax.ShapeDtypeStruct((n, H), jnp.float32),
    )(a0, a1, hm, dinv_col, b[None, :])


def _deg_to_dinv_body(d0_ref, d1_ref, gdinv_ref, pdinv_ref):
    deg = d0_ref[...] + d1_ref[...]
    gdinv_ref[...] = jax.lax.rsqrt(deg + 1.0)
    pdinv_ref[...] = jnp.where(deg > 0, 1.0 / jnp.maximum(deg, 1e-12), 0.0)


def _deg_to_dinv(d0, d1):
    n = d0.shape[0]
    bm = 512
    return pl.pallas_call(
        _deg_to_dinv_body,
        grid=(pl.cdiv(n, bm),),
        in_specs=[pl.BlockSpec((bm, 1), lambda i: (i, 0)),
                  pl.BlockSpec((bm, 1), lambda i: (i, 0))],
        out_specs=[pl.BlockSpec((bm, 1), lambda i: (i, 0)),
                   pl.BlockSpec((bm, 1), lambda i: (i, 0))],
        out_shape=[jax.ShapeDtypeStruct((n, 1), jnp.float32),
                   jax.ShapeDtypeStruct((n, 1), jnp.float32)],
    )(d0, d1)


def _score_body(x_ref, a0_ref, a1_ref, s_ref):
    s_ref[...] = jnp.sum(jnp.abs(x_ref[...] - a0_ref[...] - a1_ref[...]),
                         axis=1, keepdims=True)


def _score(x, a0, a1):
    n = x.shape[0]
    bm = 512
    return pl.pallas_call(
        _score_body,
        grid=(pl.cdiv(n, bm),),
        in_specs=[pl.BlockSpec((bm, H), lambda i: (i, 0)),
                  pl.BlockSpec((bm, H), lambda i: (i, 0)),
                  pl.BlockSpec((bm, H), lambda i: (i, 0))],
        out_specs=pl.BlockSpec((bm, 1), lambda i: (i, 0)),
        out_shape=jax.ShapeDtypeStruct((n, 1), jnp.float32),
    )(x, a0, a1)


def _topk_body(k, n, s_ref, map_ref, tanh_ref):
    rows = s_ref.shape[0]
    score = s_ref[...]
    ridx = jax.lax.broadcasted_iota(jnp.int32, (rows, H), 0)
    cidx = jax.lax.broadcasted_iota(jnp.int32, (rows, H), 1)
    flat = ridx * H + cidx
    valid = flat < n
    score = jnp.where(valid, score, -1.0)
    bits = jax.lax.bitcast_convert_type(score, jnp.uint32)
    keys = jnp.where(bits >> 31 != 0, ~bits, bits | jnp.uint32(0x80000000))
    sgn = jnp.uint32(0x80000000)
    ik = jax.lax.bitcast_convert_type(keys ^ sgn, jnp.int32)

    def body(i, carry):
        prefix, kk = carry
        bit = jnp.uint32(1) << (31 - i)
        cand = prefix | bit
        mask = ~(bit - jnp.uint32(1))
        icand = jax.lax.bitcast_convert_type(cand ^ sgn, jnp.int32)
        imasked = jax.lax.bitcast_convert_type((keys & mask) ^ sgn, jnp.int32)
        cnt = jnp.sum((imasked >= icand).astype(jnp.int32))
        take = cnt >= kk
        return (jnp.where(take, cand, prefix),
                jnp.where(take, kk, kk - cnt))

    thr, _ = jax.lax.fori_loop(0, 32, body, (jnp.uint32(0), jnp.int32(k)))
    ithr = jax.lax.bitcast_convert_type(thr ^ sgn, jnp.int32)
    gt = ik > ithr
    eq = ik == ithr
    n_gt = jnp.sum(gt.astype(jnp.int32))

    ut = (jax.lax.broadcasted_iota(jnp.int32, (H, H), 0)
          <= jax.lax.broadcasted_iota(jnp.int32, (H, H), 1)).astype(jnp.float32)
    lt_strict = (jax.lax.broadcasted_iota(jnp.int32, (rows, rows), 1)
                 < jax.lax.broadcasted_iota(jnp.int32, (rows, rows), 0)
                 ).astype(jnp.float32)

    def cumsum2d(m):
        mf = m.astype(jnp.float32)
        within = jnp.dot(mf, ut, preferred_element_type=jnp.float32)
        row_tot = jnp.sum(mf, axis=1, keepdims=True)
        excl = jnp.dot(lt_strict, jnp.broadcast_to(row_tot, (rows, H)),
                       preferred_element_type=jnp.float32)
        return (within + excl).astype(jnp.int32)

    eq_rank = cumsum2d(eq) - 1
    keep = gt | (eq & (eq_rank < (k - n_gt)))
    rank = cumsum2d(keep) - 1
    mapping = jnp.where(keep, rank, -1)

    maxk = jnp.max(ik)
    amax = jnp.min(jnp.where(ik == maxk, flat, jnp.int32(2**30)))
    r0 = jnp.sum(jnp.where(flat == amax, rank, 0))
    i0 = jnp.sum(jnp.where(keep & (rank == 0), flat, 0))
    mapping = jnp.where(flat == amax, 0,
                        jnp.where((flat == i0) & keep, r0, mapping))
    map_ref[...] = mapping
    tanh_ref[...] = jnp.tanh(score)


def _topk(score_col, k):
    n = score_col.shape[0]
    npad = ((n + H - 1) // H) * H
    rows = npad // H
    s_rs = jnp.reshape(
        jnp.pad(score_col[:, 0], (0, npad - n), constant_values=-1.0),
        (rows, H))
    mapping_rs, tanh_rs = pl.pallas_call(
        partial(_topk_body, k, n),
        in_specs=[pl.BlockSpec((rows, H), lambda: (0, 0))],
        out_specs=[pl.BlockSpec((rows, H), lambda: (0, 0)),
                   pl.BlockSpec((rows, H), lambda: (0, 0))],
        out_shape=[jax.ShapeDtypeStruct((rows, H), jnp.int32),
                   jax.ShapeDtypeStruct((rows, H), jnp.float32)],
    )(s_rs)
    mapping = jnp.reshape(mapping_rs, (npad,))[:n]
    tanhs = jnp.reshape(tanh_rs, (npad,))[:n, None]
    return mapping, tanhs


def _scale_rows_body(x_ref, t_ref, o_ref):
    o_ref[...] = x_ref[...] * t_ref[...]


def _scale_rows(x, t_col):
    n = x.shape[0]
    bm = 512
    return pl.pallas_call(
        _scale_rows_body,
        grid=(pl.cdiv(n, bm),),
        in_specs=[pl.BlockSpec((bm, H), lambda i: (i, 0)),
                  pl.BlockSpec((bm, 1), lambda i: (i, 0))],
        out_specs=pl.BlockSpec((bm, H), lambda i: (i, 0)),
        out_shape=jax.ShapeDtypeStruct((n, H), jnp.float32),
    )(x, t_col)


def _att_readout_body(nrows, xk_ref, atta_ref, attb_ref, ca_ref, cb_ref,
                      mx_ref, sm_ref):
    xk = xk_ref[...]
    ca_ref[...] = jnp.sum(xk * atta_ref[...], axis=1, keepdims=True)
    cb_ref[...] = jnp.sum(xk * attb_ref[...], axis=1, keepdims=True)
    i = pl.program_id(0)
    bm = xk.shape[0]
    valid = (i * bm + jax.lax.broadcasted_iota(jnp.int32, xk.shape, 0)) < nrows
    bmax = jnp.max(jnp.where(valid, xk, -jnp.inf), axis=0, keepdims=True)
    bsum = jnp.sum(jnp.where(valid, xk, 0.0), axis=0, keepdims=True)

    @pl.when(i == 0)
    def _():
        mx_ref[...] = bmax
        sm_ref[...] = bsum

    @pl.when(i != 0)
    def _():
        mx_ref[...] = jnp.maximum(mx_ref[...], bmax)
        sm_ref[...] = sm_ref[...] + bsum


def _att_readout(xk, att):
    kk = xk.shape[0]
    bm = 512
    atta = att[None, :H]
    attb = att[None, H:]
    return pl.pallas_call(
        partial(_att_readout_body, kk),
        grid=(pl.cdiv(kk, bm),),
        in_specs=[pl.BlockSpec((bm, H), lambda i: (i, 0)),
                  pl.BlockSpec((1, H), lambda i: (0, 0)),
                  pl.BlockSpec((1, H), lambda i: (0, 0))],
        out_specs=[pl.BlockSpec((bm, 1), lambda i: (i, 0)),
                   pl.BlockSpec((bm, 1), lambda i: (i, 0)),
                   pl.BlockSpec((1, H), lambda i: (0, 0)),
                   pl.BlockSpec((1, H), lambda i: (0, 0))],
        out_shape=[jax.ShapeDtypeStruct((kk, 1), jnp.float32),
                   jax.ShapeDtypeStruct((kk, 1), jnp.float32),
                   jax.ShapeDtypeStruct((1, H), jnp.float32),
                   jax.ShapeDtypeStruct((1, H), jnp.float32)],
    )(xk, atta, attb)


def _final_combine_body(nrows, a0_ref, a1_ref, hm_ref, dinv_ref, b_ref,
                        mx_ref, sm_ref):
    dinv = dinv_ref[...]
    h = jnp.maximum(
        a0_ref[...] + a1_ref[...] + dinv * dinv * hm_ref[...] + b_ref[...], 0.0)
    i = pl.program_id(0)
    valid = (i * h.shape[0]
             + jax.lax.broadcasted_iota(jnp.int32, h.shape, 0)) < nrows
    bmax = jnp.max(jnp.where(valid, h, -jnp.inf), axis=0, keepdims=True)
    bsum = jnp.sum(jnp.where(valid, h, 0.0), axis=0, keepdims=True)

    @pl.when(i == 0)
    def _():
        mx_ref[...] = bmax
        sm_ref[...] = bsum

    @pl.when(i != 0)
    def _():
        mx_ref[...] = jnp.maximum(mx_ref[...], bmax)
        sm_ref[...] = sm_ref[...] + bsum


def _final_combine(a0, a1, hm, dinv_col, b):
    n = hm.shape[0]
    bm = 512
    return pl.pallas_call(
        partial(_final_combine_body, n),
        grid=(pl.cdiv(n, bm),),
        in_specs=[pl.BlockSpec((bm, H), lambda i: (i, 0)),
                  pl.BlockSpec((bm, H), lambda i: (i, 0)),
                  pl.BlockSpec((bm, H), lambda i: (i, 0)),
                  pl.BlockSpec((bm, 1), lambda i: (i, 0)),
                  pl.BlockSpec((1, H), lambda i: (0, 0))],
        out_specs=[pl.BlockSpec((1, H), lambda i: (0, 0)),
                   pl.BlockSpec((1, H), lambda i: (0, 0))],
        out_shape=[jax.ShapeDtypeStruct((1, H), jnp.float32),
                   jax.ShapeDtypeStruct((1, H), jnp.float32)],
    )(a0, a1, hm, dinv_col, b[None, :])


def _head_body(k1_inv, k2_inv, k3_inv,
               mx1_ref, sm1_ref, mx2_ref, sm2_ref, mx3_ref, sm3_ref,
               wf1a_ref, wf1b_ref, bf1_ref, wf2_ref, bf2_ref, wf3_ref,
               bf3_ref, o_ref):
    ra = (jnp.maximum(mx1_ref[...], 0.0) + jnp.maximum(mx2_ref[...], 0.0)
          + jnp.maximum(mx3_ref[...], 0.0))
    rb = (jnp.maximum(sm1_ref[...] * k1_inv, 0.0)
          + jnp.maximum(sm2_ref[...] * k2_inv, 0.0)
          + jnp.maximum(sm3_ref[...] * k3_inv, 0.0))
    o = jnp.maximum(
        jnp.dot(ra, wf1a_ref[...], preferred_element_type=jnp.float32)
        + jnp.dot(rb, wf1b_ref[...], preferred_element_type=jnp.float32)
        + bf1_ref[...], 0.0)
    o = jnp.maximum(
        jnp.dot(o, wf2_ref[...], preferred_element_type=jnp.float32)
        + bf2_ref[...], 0.0)
    lg = jnp.dot(o, wf3_ref[...], preferred_element_type=jnp.float32) \
        + bf3_ref[...]
    lane = jax.lax.broadcasted_iota(jnp.int32, (1, H), 1)
    lvalid = lane < 2
    m = jnp.max(jnp.where(lvalid, lg, -jnp.inf))
    s = jnp.sum(jnp.where(lvalid, jnp.exp(lg - m), 0.0))
    o_ref[...] = lg - m - jnp.log(s)


def _head(mx1, sm1, mx2, sm2, mx3, sm3, W_fc1, b_fc1, W_fc2, b_fc2,
          W_fc3, b_fc3):
    wf1a = W_fc1[:H]
    wf1b = W_fc1[H:]
    wf2 = jnp.zeros((H, H), jnp.float32).at[:, :H // 2].set(W_fc2)
    bf2 = jnp.zeros((1, H), jnp.float32).at[0, :H // 2].set(b_fc2)
    wf3 = jnp.zeros((H, H), jnp.float32).at[:H // 2, :2].set(W_fc3)
    bf3 = jnp.zeros((1, H), jnp.float32).at[0, :2].set(b_fc3)
    out = pl.pallas_call(
        partial(_head_body, 1.0 / K1, 1.0 / K2, 1.0 / K2),
        in_specs=[pl.BlockSpec((1, H), lambda: (0, 0))] * 6 +
                 [pl.BlockSpec((H, H), lambda: (0, 0)),
                  pl.BlockSpec((H, H), lambda: (0, 0)),
                  pl.BlockSpec((1, H), lambda: (0, 0)),
                  pl.BlockSpec((H, H), lambda: (0, 0)),
                  pl.BlockSpec((1, H), lambda: (0, 0)),
                  pl.BlockSpec((H, H), lambda: (0, 0)),
                  pl.BlockSpec((1, H), lambda: (0, 0))],
        out_specs=pl.BlockSpec((1, H), lambda: (0, 0)),
        out_shape=jax.ShapeDtypeStruct((1, H), jnp.float32),
    )(mx1, sm1, mx2, sm2, mx3, sm3, wf1a, wf1b, b_fc1[None, :], wf2, bf2,
      wf3, bf3)
    return out[:, :2]


def _sc_deg(dst, n):
    ones = jnp.ones((E,), jnp.float32)
    deg = jax.ops.segment_sum(ones, dst, num_segments=n)
    return deg[:, None], jnp.zeros((n, 1), jnp.float32)


def _sc_agg(h, src, dst, coef_src_tab, w, coef_dst_tab, n):
    c = coef_src_tab[src] * w * coef_dst_tab[dst]
    out = jax.ops.segment_sum(c[:, None] * h[src], dst, num_segments=n)
    return out, jnp.zeros_like(out)


def _sc_xk_scatter(xs, mapping, k):
    tgt = jnp.where(mapping >= 0, mapping, k)
    xk = jnp.zeros((k + 1, H), jnp.float32).at[tgt].set(xs)
    return xk[:k]


def _sc_attn1(src, dst, w, mapping, ca, cb, k):
    ms = mapping[src]
    md = mapping[dst]
    kp = (ms >= 0) & (md >= 0)
    s2c = jnp.where(kp, ms, 0)
    d2c = jnp.where(kp, md, 0)
    wk = jnp.where(kp, w, 0.0)
    e = jnp.maximum(ca[s2c] + cb[d2c], 0.0) + LAMB * wk
    ee = jnp.where(kp, jnp.exp(e), 0.0)
    denom = jax.ops.segment_sum(ee, d2c, num_segments=k)
    return s2c, d2c, ee, denom[:, None], jnp.zeros((k, 1), jnp.float32)


def _sc_attn2(ee, d2c, denom):
    return ee / jnp.maximum(denom[d2c, 0], 1e-16)


def _pool_stage(h, src, dst, w, att, pdinv_col, k, n):
    ones_tab = jnp.ones((n,), jnp.float32)
    a0, a1 = _sc_agg(h, src, dst, ones_tab, w, pdinv_col[:, 0], n)
    score_col = _score(h, a0, a1)
    mapping, tanhs = _topk(score_col, k)
    xs = _scale_rows(h, tanhs)
    xk = _sc_xk_scatter(xs, mapping, k)
    ca, cb, rmax, rsum = _att_readout(xk, att)
    s2c, d2c, ee, den0, den1 = _sc_attn1(src, dst, w, mapping, ca[:, 0],
                                         cb[:, 0], k)
    denom = den0 + den1
    wnew = _sc_attn2(ee, d2c, denom)
    deg_next = (denom > 1e-16).astype(jnp.float32)
    gdinv = jax.lax.rsqrt(deg_next + 1.0)
    pdinv = deg_next
    return xk, s2c, d2c, wnew, gdinv, pdinv, rmax, rsum


def kernel(x, edge_index, batch, W_lin1, b_lin1, W_lin2, b_lin2, W_conv1,
           b_conv1, W_conv2, b_conv2, W_conv3, b_conv3, att1, att2, W_fc1,
           b_fc1, W_fc2, b_fc2, W_fc3, b_fc3):
    src = edge_index[0].astype(jnp.int32)
    dst = edge_index[1].astype(jnp.int32)
    w0 = jnp.ones((E,), jnp.float32)

    hm1 = _front(x, W_lin1, b_lin1, W_lin2, b_lin2, W_conv1, b_conv1)
    d0, d1 = _sc_deg(dst, N0)
    gdinv1, pdinv1 = _deg_to_dinv(d0, d1)
    a0, a1 = _sc_agg(hm1, src, dst, gdinv1[:, 0], w0, gdinv1[:, 0], N0)
    h1 = _gcn_combine(a0, a1, hm1, gdinv1, b_conv1)

    xk1, s1, d1e, w1, gdinv2, pdinv2, mx1, sm1 = _pool_stage(
        h1, src, dst, w0, att1, pdinv1, K1, N0)

    hm2 = _matmul(xk1, W_conv2)
    a0, a1 = _sc_agg(hm2, s1, d1e, gdinv2[:, 0], w1, gdinv2[:, 0], K1)
    h2 = _gcn_combine(a0, a1, hm2, gdinv2, b_conv2)

    xk2, s2, d2e, w2, gdinv3, pdinv3, mx2, sm2 = _pool_stage(
        h2, s1, d1e, w1, att2, pdinv2, K2, K1)

    hm3 = _matmul(xk2, W_conv3)
    a0, a1 = _sc_agg(hm3, s2, d2e, gdinv3[:, 0], w2, gdinv3[:, 0], K2)
    mx3, sm3 = _final_combine(a0, a1, hm3, gdinv3, b_conv3)

    return _head(mx1, sm1, mx2, sm2, mx3, sm3, W_fc1, b_fc1, W_fc2, b_fc2,
                 W_fc3, b_fc3)


# speedup vs baseline: 3.4141x; 3.3701x over previous
from functools import partial

import jax
import jax.numpy as jnp
from jax import lax
from jax.experimental import pallas as pl
from jax.experimental.pallas import tpu as pltpu
from jax.experimental.pallas import tpu_sc as plsc

N0 = 10000
NP = 10240
E = 320000
EP = 327680
ER = EP // 128
AA = 21
H = 128
LAMB = 1.0
K1 = 5000
K2 = 2500
DEAD = NP - 1

_NC, _NS, _L = 2, 16, 16
_NW = _NC * _NS
_ERW = ER // _NW
_RCH = 4
_NCH = _ERW // _RCH
_SLAB = NP // _NS


def _mesh():
    return plsc.VectorSubcoreMesh(core_axis_name="c", subcore_axis_name="s")


def _zero_rows(zref):
    rows, cols = zref.shape
    nv = cols // _L

    def zb(i, _):
        zref[i // nv, pl.ds((i % nv) * _L, _L)] = jnp.zeros((_L,), jnp.float32)
        return 0

    lax.fori_loop(0, rows * nv, zb, 0)


def _zero_vec(zref):
    n = zref.shape[0]

    def zb(i, _):
        zref[pl.ds(i * _L, _L)] = jnp.zeros((_L,), jnp.float32)
        return 0

    lax.fori_loop(0, n // _L, zb, 0)


def _sc_deg(dst2):
    @partial(
        pl.kernel, mesh=_mesh(),
        compiler_params=pltpu.CompilerParams(needs_layout_passes=False),
        out_type=jax.ShapeDtypeStruct((_NC, NP), jnp.float32),
        scratch_types=[
            pltpu.VMEM_SHARED((NP,), jnp.float32),
            pltpu.VMEM((_RCH, 128), jnp.int32),
            pltpu.VMEM((128,), jnp.float32),
            pltpu.VMEM((_SLAB,), jnp.float32),
        ],
    )
    def k(dst_h, out_h, deg_sh, dstv, onesv, zv):
        cid = lax.axis_index("c")
        sid = lax.axis_index("s")
        wid = sid * _NC + cid
        _zero_vec(zv)

        def ob(i, _):
            onesv[pl.ds(i * _L, _L)] = jnp.ones((_L,), jnp.float32)
            return 0

        lax.fori_loop(0, 128 // _L, ob, 0)
        pltpu.sync_copy(zv, deg_sh.at[pl.ds(sid * _SLAB, _SLAB)])
        plsc.subcore_barrier()

        def chunk(g, _):
            erow = wid * _ERW + g * _RCH
            pltpu.sync_copy(dst_h.at[pl.ds(erow, _RCH)], dstv)
            for r in range(_RCH):
                pltpu.sync_copy(onesv, deg_sh.at[dstv.at[r]], add=True)
            return 0

        lax.fori_loop(0, _NCH, chunk, 0)
        plsc.subcore_barrier()
        pltpu.sync_copy(deg_sh.at[pl.ds(sid * _SLAB, _SLAB)],
                        out_h.at[cid, pl.ds(sid * _SLAB, _SLAB)])

    return k(dst2)


def _sc_agg(T, src2, dst2, w2, tab, mode):
    both = mode == "both"

    @partial(
        pl.kernel, mesh=_mesh(),
        compiler_params=pltpu.CompilerParams(needs_layout_passes=False),
        out_type=jax.ShapeDtypeStruct((_NC, NP, H), jnp.float32),
        scratch_types=[
            pltpu.VMEM_SHARED((NP, H), jnp.float32),
            pltpu.VMEM((NP,), jnp.float32),
            pltpu.VMEM((1, 128), jnp.int32),
            pltpu.VMEM((1, 128), jnp.int32),
            pltpu.VMEM((1, 128), jnp.float32),
            pltpu.VMEM((1, 128), jnp.float32),
            pltpu.VMEM((128, H), jnp.float32),
            pltpu.VMEM((64, H), jnp.float32),
            pltpu.SemaphoreType.DMA,
        ],
    )
    def k(T_h, src_h, dst_h, w_h, tab_h, out_h,
          acc_sh, tabv, srcv, dstv, wv, coefv, rows, zr, sem):
        cid = lax.axis_index("c")
        sid = lax.axis_index("s")
        wid = sid * _NC + cid
        _zero_rows(zr)

        def zslab(i, _):
            pltpu.sync_copy(zr, acc_sh.at[pl.ds(sid * _SLAB + i * 64, 64)])
            return 0

        lax.fori_loop(0, _SLAB // 64, zslab, 0)
        pltpu.sync_copy(tab_h, tabv)
        plsc.subcore_barrier()

        def chunk(g, _):
            erow = wid * _ERW + g
            pltpu.sync_copy(src_h.at[pl.ds(erow, 1)], srcv)
            pltpu.sync_copy(dst_h.at[pl.ds(erow, 1)], dstv)
            pltpu.sync_copy(w_h.at[pl.ds(erow, 1)], wv)
            pltpu.async_copy(T_h.at[srcv.at[0]], rows, sem).wait()
            for j in range(128 // _L):
                sl = pl.ds(j * _L, _L)
                d16 = dstv[0, sl]
                c = wv[0, sl] * plsc.load_gather(tabv, [d16])
                if both:
                    s16 = srcv[0, sl]
                    c = c * plsc.load_gather(tabv, [s16])
                coefv[0, sl] = c

            def ebody(e, _):
                ci = plsc.load_gather(
                    coefv, [jnp.zeros((_L,), jnp.int32),
                            jnp.full((_L,), e, jnp.int32)])
                for j in range(H // _L):
                    sl = pl.ds(j * _L, _L)
                    rows[e, sl] = rows[e, sl] * ci
                return 0

            lax.fori_loop(0, 128, ebody, 0)
            pltpu.sync_copy(rows, acc_sh.at[dstv.at[0]], add=True)
            return 0

        lax.fori_loop(0, _ERW, chunk, 0)
        plsc.subcore_barrier()

        def drain(i, _):
            row0 = sid * _SLAB + i * 64
            pltpu.sync_copy(acc_sh.at[pl.ds(row0, 64)],
                            out_h.at[cid, pl.ds(row0, 64)])
            return 0

        lax.fori_loop(0, _SLAB // 64, drain, 0)

    return k(T, src2, dst2, w2, tab)


def _sc_attn1a(src2, dst2, ca, cb, k1t):
    @partial(
        pl.kernel, mesh=_mesh(),
        compiler_params=pltpu.CompilerParams(needs_layout_passes=False),
        out_type=[jax.ShapeDtypeStruct((ER, 128), jnp.float32),
                  jax.ShapeDtypeStruct((_NC, NP), jnp.float32)],
        scratch_types=[
            pltpu.VMEM_SHARED((NP,), jnp.float32),
            pltpu.VMEM((NP,), jnp.float32),
            pltpu.VMEM((NP,), jnp.float32),
            pltpu.VMEM((NP,), jnp.float32),
            pltpu.VMEM((_RCH, 128), jnp.int32),
            pltpu.VMEM((_RCH, 128), jnp.int32),
            pltpu.VMEM((_RCH, 128), jnp.float32),
            pltpu.VMEM((_SLAB,), jnp.float32),
        ],
    )
    def k(src_h, dst_h, ca_h, cb_h, k1_h, ee_h, den_h,
          den_sh, cav, cbv, k1v, srcv, dstv, eev, zv):
        cid = lax.axis_index("c")
        sid = lax.axis_index("s")
        wid = sid * _NC + cid
        _zero_vec(zv)
        pltpu.sync_copy(zv, den_sh.at[pl.ds(sid * _SLAB, _SLAB)])
        pltpu.sync_copy(ca_h, cav)
        pltpu.sync_copy(cb_h, cbv)
        pltpu.sync_copy(k1_h, k1v)
        plsc.subcore_barrier()

        def chunk(g, _):
            erow = wid * _ERW + g * _RCH
            pltpu.sync_copy(src_h.at[pl.ds(erow, _RCH)], srcv)
            pltpu.sync_copy(dst_h.at[pl.ds(erow, _RCH)], dstv)
            for r in range(_RCH):
                for j in range(128 // _L):
                    sl = pl.ds(j * _L, _L)
                    s16 = srcv[r, sl]
                    d16 = dstv[r, sl]
                    kp = ((plsc.load_gather(k1v, [s16]) > 0.5)
                          & (plsc.load_gather(k1v, [d16]) > 0.5))
                    ev = (jnp.maximum(plsc.load_gather(cav, [s16])
                                      + plsc.load_gather(cbv, [d16]), 0.0)
                          + LAMB)
                    eev[r, sl] = jnp.where(kp, jnp.exp(ev), 0.0)
            pltpu.sync_copy(eev, ee_h.at[pl.ds(erow, _RCH)])
            for r in range(_RCH):
                pltpu.sync_copy(eev.at[r], den_sh.at[dstv.at[r]], add=True)
            return 0

        lax.fori_loop(0, _NCH, chunk, 0)
        plsc.subcore_barrier()
        pltpu.sync_copy(den_sh.at[pl.ds(sid * _SLAB, _SLAB)],
                        den_h.at[cid, pl.ds(sid * _SLAB, _SLAB)])

    return k(src2, dst2, ca, cb, k1t)


def _sc_attn_norm(ee2, dst2, den):
    @partial(
        pl.kernel, mesh=_mesh(),
        compiler_params=pltpu.CompilerParams(needs_layout_passes=False),
        out_type=jax.ShapeDtypeStruct((ER, 128), jnp.float32),
        scratch_types=[
            pltpu.VMEM((NP,), jnp.float32),
            pltpu.VMEM((NP,), jnp.float32),
            pltpu.VMEM((_RCH, 128), jnp.int32),
            pltpu.VMEM((_RCH, 128), jnp.float32),
            pltpu.VMEM((_RCH, 128), jnp.float32),
        ],
    )
    def k(ee_h, dst_h, den_h, w_h, denv, den2v, dstv, eev, wv):
        cid = lax.axis_index("c")
        sid = lax.axis_index("s")
        wid = sid * _NC + cid
        pltpu.sync_copy(den_h.at[0], denv)
        pltpu.sync_copy(den_h.at[1], den2v)

        def ab(i, _):
            sl = pl.ds(i * _L, _L)
            denv[sl] = jnp.maximum(denv[sl] + den2v[sl], 1e-16)
            return 0

        lax.fori_loop(0, NP // _L, ab, 0)

        def chunk(g, _):
            erow = wid * _ERW + g * _RCH
            pltpu.sync_copy(dst_h.at[pl.ds(erow, _RCH)], dstv)
            pltpu.sync_copy(ee_h.at[pl.ds(erow, _RCH)], eev)
            for r in range(_RCH):
                for j in range(128 // _L):
                    sl = pl.ds(j * _L, _L)
                    d16 = dstv[r, sl]
                    wv[r, sl] = eev[r, sl] / plsc.load_gather(denv, [d16])
            pltpu.sync_copy(wv, w_h.at[pl.ds(erow, _RCH)])
            return 0

        lax.fori_loop(0, _NCH, chunk, 0)

    return k(ee2, dst2, den)


def _sc_attn2a(src2, dst2, w1, ca, cb, k1t, k2t, a1v):
    @partial(
        pl.kernel, mesh=_mesh(),
        compiler_params=pltpu.CompilerParams(needs_layout_passes=False),
        out_type=[jax.ShapeDtypeStruct((ER, 128), jnp.int32),
                  jax.ShapeDtypeStruct((ER, 128), jnp.int32),
                  jax.ShapeDtypeStruct((ER, 128), jnp.float32),
                  jax.ShapeDtypeStruct((_NC, NP), jnp.float32)],
        scratch_types=[
            pltpu.VMEM_SHARED((NP,), jnp.float32),
            pltpu.VMEM((NP,), jnp.float32),
            pltpu.VMEM((NP,), jnp.float32),
            pltpu.VMEM((NP,), jnp.float32),
            pltpu.VMEM((NP,), jnp.float32),
            pltpu.VMEM((16,), jnp.int32),
            pltpu.VMEM((_RCH, 128), jnp.int32),
            pltpu.VMEM((_RCH, 128), jnp.int32),
            pltpu.VMEM((_RCH, 128), jnp.float32),
            pltpu.VMEM((_RCH, 128), jnp.int32),
            pltpu.VMEM((_RCH, 128), jnp.int32),
            pltpu.VMEM((_RCH, 128), jnp.float32),
            pltpu.VMEM((_SLAB,), jnp.float32),
        ],
    )
    def k(src_h, dst_h, w_h, ca_h, cb_h, k1_h, k2_h, a1_h,
          sh_h, dh_h, ee_h, den_h,
          den_sh, cav, cbv, k1v, k2v, a1vm, srcv, dstv, wv, shv, dhv, eev,
          zv):
        cid = lax.axis_index("c")
        sid = lax.axis_index("s")
        wid = sid * _NC + cid
        _zero_vec(zv)
        pltpu.sync_copy(zv, den_sh.at[pl.ds(sid * _SLAB, _SLAB)])
        pltpu.sync_copy(ca_h, cav)
        pltpu.sync_copy(cb_h, cbv)
        pltpu.sync_copy(k1_h, k1v)
        pltpu.sync_copy(k2_h, k2v)
        pltpu.sync_copy(a1_h, a1vm)
        plsc.subcore_barrier()

        def chunk(g, _):
            erow = wid * _ERW + g * _RCH
            pltpu.sync_copy(src_h.at[pl.ds(erow, _RCH)], srcv)
            pltpu.sync_copy(dst_h.at[pl.ds(erow, _RCH)], dstv)
            pltpu.sync_copy(w_h.at[pl.ds(erow, _RCH)], wv)
            a1l = a1vm[...]
            dead = jnp.full((_L,), DEAD, jnp.int32)
            for r in range(_RCH):
                for j in range(128 // _L):
                    sl = pl.ds(j * _L, _L)
                    s16 = srcv[r, sl]
                    d16 = dstv[r, sl]
                    valid = s16 != dead
                    kp1 = ((plsc.load_gather(k1v, [s16]) > 0.5)
                           & (plsc.load_gather(k1v, [d16]) > 0.5))
                    sh = jnp.where(valid, jnp.where(kp1, s16, a1l), dead)
                    dh = jnp.where(valid, jnp.where(kp1, d16, a1l), dead)
                    kp2 = ((plsc.load_gather(k2v, [sh]) > 0.5)
                           & (plsc.load_gather(k2v, [dh]) > 0.5))
                    wk = jnp.where(kp2, wv[r, sl], 0.0)
                    ev = (jnp.maximum(plsc.load_gather(cav, [sh])
                                      + plsc.load_gather(cbv, [dh]), 0.0)
                          + LAMB * wk)
                    shv[r, sl] = sh
                    dhv[r, sl] = dh
                    eev[r, sl] = jnp.where(kp2, jnp.exp(ev), 0.0)
            pltpu.sync_copy(shv, sh_h.at[pl.ds(erow, _RCH)])
            pltpu.sync_copy(dhv, dh_h.at[pl.ds(erow, _RCH)])
            pltpu.sync_copy(eev, ee_h.at[pl.ds(erow, _RCH)])
            for r in range(_RCH):
                pltpu.sync_copy(eev.at[r], den_sh.at[dhv.at[r]], add=True)
            return 0

        lax.fori_loop(0, _NCH, chunk, 0)
        plsc.subcore_barrier()
        pltpu.sync_copy(den_sh.at[pl.ds(sid * _SLAB, _SLAB)],
                        den_h.at[cid, pl.ds(sid * _SLAB, _SLAB)])

    return k(src2, dst2, w1, ca, cb, k1t, k2t, a1v)


def _front_body(x_ref, w2p_ref, b2p_ref, wc1a_ref, w1p_ref, b1p_ref,
                wc1b_ref, hm_ref):
    x = x_ref[...]
    xb = jnp.maximum(jnp.dot(x, w2p_ref[...],
                             preferred_element_type=jnp.float32)
                     + b2p_ref[...], 0.0)
    xa = jnp.maximum(jnp.dot(x, w1p_ref[...],
                             preferred_element_type=jnp.float32)
                     + b1p_ref[...], 0.0)
    hm_ref[...] = (jnp.dot(xb, wc1a_ref[...], preferred_element_type=jnp.float32)
                   + jnp.dot(xa, wc1b_ref[...], preferred_element_type=jnp.float32))


def _front(x, W_lin1, b_lin1, W_lin2, b_lin2, W_conv1):
    W2p = jnp.zeros((H, H), jnp.float32).at[:AA, :AA].set(W_lin2)
    b2p = jnp.zeros((1, H), jnp.float32).at[0, :AA].set(b_lin2)
    Wc1a = jnp.zeros((H, H), jnp.float32).at[:AA, :].set(W_conv1[:AA])
    W1p = jnp.zeros((H, H), jnp.float32).at[AA:, :].set(W_lin1)
    b1p = jnp.broadcast_to(b_lin1[None, :], (1, H))
    Wc1b = W_conv1[AA:]
    bm = 512
    return pl.pallas_call(
        _front_body,
        grid=(NP // bm,),
        in_specs=[pl.BlockSpec((bm, H), lambda i: (i, 0))] +
                 [pl.BlockSpec((H, H), lambda i: (0, 0)),
                  pl.BlockSpec((1, H), lambda i: (0, 0)),
                  pl.BlockSpec((H, H), lambda i: (0, 0)),
                  pl.BlockSpec((H, H), lambda i: (0, 0)),
                  pl.BlockSpec((1, H), lambda i: (0, 0)),
                  pl.BlockSpec((H, H), lambda i: (0, 0))],
        out_specs=pl.BlockSpec((bm, H), lambda i: (i, 0)),
        out_shape=jax.ShapeDtypeStruct((NP, H), jnp.float32),
    )(x, W2p, b2p, Wc1a, W1p, b1p, Wc1b)


def _matmul_body(x_ref, w_ref, o_ref):
    o_ref[...] = jnp.dot(x_ref[...], w_ref[...],
                         preferred_element_type=jnp.float32)


def _matmul(x, w):
    bm = 512
    return pl.pallas_call(
        _matmul_body,
        grid=(NP // bm,),
        in_specs=[pl.BlockSpec((bm, H), lambda i: (i, 0)),
                  pl.BlockSpec((H, H), lambda i: (0, 0))],
        out_specs=pl.BlockSpec((bm, H), lambda i: (i, 0)),
        out_shape=jax.ShapeDtypeStruct((NP, H), jnp.float32),
    )(x, w)


def _gcn_combine_body(a0_ref, a1_ref, hm_ref, dinv_ref, b_ref, m_ref, o_ref):
    dinv = dinv_ref[...]
    o_ref[...] = m_ref[...] * jnp.maximum(
        a0_ref[...] + a1_ref[...] + dinv * dinv * hm_ref[...] + b_ref[...], 0.0)


def _gcn_combine(a0, a1, hm, dinv_col, b, mask_col):
    bm = 512
    return pl.pallas_call(
        _gcn_combine_body,
        grid=(NP // bm,),
        in_specs=[pl.BlockSpec((bm, H), lambda i: (i, 0)),
                  pl.BlockSpec((bm, H), lambda i: (i, 0)),
                  pl.BlockSpec((bm, H), lambda i: (i, 0)),
                  pl.BlockSpec((bm, 1), lambda i: (i, 0)),
                  pl.BlockSpec((1, H), lambda i: (0, 0)),
                  pl.BlockSpec((bm, 1), lambda i: (i, 0))],
        out_specs=pl.BlockSpec((bm, H), lambda i: (i, 0)),
        out_shape=jax.ShapeDtypeStruct((NP, H), jnp.float32),
    )(a0, a1, hm, dinv_col, b[None, :], mask_col)


def _deg_to_dinv_body(d0_ref, d1_ref, gdinv_ref, pdinv_ref):
    deg = d0_ref[...] + d1_ref[...]
    gdinv_ref[...] = jax.lax.rsqrt(deg + 1.0)
    pdinv_ref[...] = jnp.where(deg > 0, 1.0 / jnp.maximum(deg, 1e-12), 0.0)


def _deg_to_dinv(d0, d1):
    bm = 512
    return pl.pallas_call(
        _deg_to_dinv_body,
        grid=(NP // bm,),
        in_specs=[pl.BlockSpec((bm, 1), lambda i: (i, 0)),
                  pl.BlockSpec((bm, 1), lambda i: (i, 0))],
        out_specs=[pl.BlockSpec((bm, 1), lambda i: (i, 0)),
                   pl.BlockSpec((bm, 1), lambda i: (i, 0))],
        out_shape=[jax.ShapeDtypeStruct((NP, 1), jnp.float32),
                   jax.ShapeDtypeStruct((NP, 1), jnp.float32)],
    )(d0, d1)


def _den_to_dinv_body(d0_ref, d1_ref, gdinv_ref, pdinv_ref):
    deg = ((d0_ref[...] + d1_ref[...]) > 1e-16).astype(jnp.float32)
    gdinv_ref[...] = jax.lax.rsqrt(deg + 1.0)
    pdinv_ref[...] = deg


def _den_to_dinv(d0, d1):
    bm = 512
    return pl.pallas_call(
        _den_to_dinv_body,
        grid=(NP // bm,),
        in_specs=[pl.BlockSpec((bm, 1), lambda i: (i, 0)),
                  pl.BlockSpec((bm, 1), lambda i: (i, 0))],
        out_specs=[pl.BlockSpec((bm, 1), lambda i: (i, 0)),
                   pl.BlockSpec((bm, 1), lambda i: (i, 0))],
        out_shape=[jax.ShapeDtypeStruct((NP, 1), jnp.float32),
                   jax.ShapeDtypeStruct((NP, 1), jnp.float32)],
    )(d0, d1)


def _score_body(x_ref, a0_ref, a1_ref, s_ref):
    s_ref[...] = jnp.sum(jnp.abs(x_ref[...] - a0_ref[...] - a1_ref[...]),
                         axis=1, keepdims=True)


def _score(x, a0, a1):
    bm = 512
    return pl.pallas_call(
        _score_body,
        grid=(NP // bm,),
        in_specs=[pl.BlockSpec((bm, H), lambda i: (i, 0)),
                  pl.BlockSpec((bm, H), lambda i: (i, 0)),
                  pl.BlockSpec((bm, H), lambda i: (i, 0))],
        out_specs=pl.BlockSpec((bm, 1), lambda i: (i, 0)),
        out_shape=jax.ShapeDtypeStruct((NP, 1), jnp.float32),
    )(x, a0, a1)


def _topk_body(k, s_ref, v_ref, keep_ref, scale_ref, amax_ref):
    rows = s_ref.shape[0]
    score = s_ref[...]
    valid = v_ref[...] > 0.5
    ridx = jax.lax.broadcasted_iota(jnp.int32, (rows, H), 0)
    cidx = jax.lax.broadcasted_iota(jnp.int32, (rows, H), 1)
    flat = ridx * H + cidx
    bits = jax.lax.bitcast_convert_type(score, jnp.uint32)
    keys = jnp.where(bits >> 31 != 0, ~bits, bits | jnp.uint32(0x80000000))
    keys = jnp.where(valid, keys, jnp.uint32(0))
    sgn = jnp.uint32(0x80000000)
    ik = jax.lax.bitcast_convert_type(keys ^ sgn, jnp.int32)

    def body(i, carry):
        prefix, kk = carry
        bit = jnp.uint32(1) << (31 - i)
        cand = prefix | bit
        mask = ~(bit - jnp.uint32(1))
        icand = jax.lax.bitcast_convert_type(cand ^ sgn, jnp.int32)
        imasked = jax.lax.bitcast_convert_type((keys & mask) ^ sgn, jnp.int32)
        cnt = jnp.sum((imasked >= icand).astype(jnp.int32))
        take = cnt >= kk
        return (jnp.where(take, cand, prefix),
                jnp.where(take, kk, kk - cnt))

    thr, _ = jax.lax.fori_loop(0, 32, body, (jnp.uint32(0), jnp.int32(k)))
    ithr = jax.lax.bitcast_convert_type(thr ^ sgn, jnp.int32)
    gt = ik > ithr
    eq = valid & (ik == ithr)
    n_gt = jnp.sum(gt.astype(jnp.int32))

    ut = (jax.lax.broadcasted_iota(jnp.int32, (H, H), 0)
          <= jax.lax.broadcasted_iota(jnp.int32, (H, H), 1)).astype(jnp.float32)
    lt_strict = (jax.lax.broadcasted_iota(jnp.int32, (rows, rows), 1)
                 < jax.lax.broadcasted_iota(jnp.int32, (rows, rows), 0)
                 ).astype(jnp.float32)

    def cumsum2d(m):
        mf = m.astype(jnp.float32)
        within = jnp.dot(mf, ut, preferred_element_type=jnp.float32)
        row_tot = jnp.sum(mf, axis=1, keepdims=True)
        excl = jnp.dot(lt_strict, jnp.broadcast_to(row_tot, (rows, H)),
                       preferred_element_type=jnp.float32)
        return (within + excl).astype(jnp.int32)

    eq_rank = cumsum2d(eq) - 1
    keep = gt | (eq & (eq_rank < (k - n_gt)))
    maxk = jnp.max(ik)
    amax = jnp.min(jnp.where(ik == maxk, flat, jnp.int32(2**30)))
    keep_ref[...] = keep.astype(jnp.float32)
    scale_ref[...] = jnp.where(keep, jnp.tanh(score), 0.0)
    amax_ref[...] = jnp.full((1, H), amax, jnp.int32)


def _topk(score_col, valid_col, k):
    rows = NP // H
    s_rs = jnp.reshape(score_col[:, 0], (rows, H))
    v_rs = jnp.reshape(valid_col[:, 0], (rows, H))
    keep_rs, scale_rs, amax_o = pl.pallas_call(
        partial(_topk_body, k),
        in_specs=[pl.BlockSpec((rows, H), lambda: (0, 0)),
                  pl.BlockSpec((rows, H), lambda: (0, 0))],
        out_specs=[pl.BlockSpec((rows, H), lambda: (0, 0)),
                   pl.BlockSpec((rows, H), lambda: (0, 0)),
                   pl.BlockSpec((1, H), lambda: (0, 0))],
        out_shape=[jax.ShapeDtypeStruct((rows, H), jnp.float32),
                   jax.ShapeDtypeStruct((rows, H), jnp.float32),
                   jax.ShapeDtypeStruct((1, H), jnp.int32)],
    )(s_rs, v_rs)
    keep = jnp.reshape(keep_rs, (NP, 1))
    scale = jnp.reshape(scale_rs, (NP, 1))
    return keep, scale, amax_o[0, :16]


def _scale_rows_body(x_ref, t_ref, o_ref):
    o_ref[...] = x_ref[...] * t_ref[...]


def _scale_rows(x, t_col):
    bm = 512
    return pl.pallas_call(
        _scale_rows_body,
        grid=(NP // bm,),
        in_specs=[pl.BlockSpec((bm, H), lambda i: (i, 0)),
                  pl.BlockSpec((bm, 1), lambda i: (i, 0))],
        out_specs=pl.BlockSpec((bm, H), lambda i: (i, 0)),
        out_shape=jax.ShapeDtypeStruct((NP, H), jnp.float32),
    )(x, t_col)


def _att_readout_body(xk_ref, atta_ref, attb_ref, ca_ref, cb_ref, mx_ref,
                      sm_ref):
    xk = xk_ref[...]
    ca_ref[...] = jnp.sum(xk * atta_ref[...], axis=1, keepdims=True)
    cb_ref[...] = jnp.sum(xk * attb_ref[...], axis=1, keepdims=True)
    i = pl.program_id(0)
    bmax = jnp.max(xk, axis=0, keepdims=True)
    bsum = jnp.sum(xk, axis=0, keepdims=True)

    @pl.when(i == 0)
    def _():
        mx_ref[...] = bmax
        sm_ref[...] = bsum

    @pl.when(i != 0)
    def _():
        mx_ref[...] = jnp.maximum(mx_ref[...], bmax)
        sm_ref[...] = sm_ref[...] + bsum


def _att_readout(xk, att):
    bm = 512
    atta = att[None, :H]
    attb = att[None, H:]
    return pl.pallas_call(
        _att_readout_body,
        grid=(NP // bm,),
        in_specs=[pl.BlockSpec((bm, H), lambda i: (i, 0)),
                  pl.BlockSpec((1, H), lambda i: (0, 0)),
                  pl.BlockSpec((1, H), lambda i: (0, 0))],
        out_specs=[pl.BlockSpec((bm, 1), lambda i: (i, 0)),
                   pl.BlockSpec((bm, 1), lambda i: (i, 0)),
                   pl.BlockSpec((1, H), lambda i: (0, 0)),
                   pl.BlockSpec((1, H), lambda i: (0, 0))],
        out_shape=[jax.ShapeDtypeStruct((NP, 1), jnp.float32),
                   jax.ShapeDtypeStruct((NP, 1), jnp.float32),
                   jax.ShapeDtypeStruct((1, H), jnp.float32),
                   jax.ShapeDtypeStruct((1, H), jnp.float32)],
    )(xk, atta, attb)


def _final_combine_body(a0_ref, a1_ref, hm_ref, dinv_ref, b_ref, m_ref,
                        mx_ref, sm_ref):
    dinv = dinv_ref[...]
    h = m_ref[...] * jnp.maximum(
        a0_ref[...] + a1_ref[...] + dinv * dinv * hm_ref[...] + b_ref[...], 0.0)
    i = pl.program_id(0)
    bmax = jnp.max(h, axis=0, keepdims=True)
    bsum = jnp.sum(h, axis=0, keepdims=True)

    @pl.when(i == 0)
    def _():
        mx_ref[...] = bmax
        sm_ref[...] = bsum

    @pl.when(i != 0)
    def _():
        mx_ref[...] = jnp.maximum(mx_ref[...], bmax)
        sm_ref[...] = sm_ref[...] + bsum


def _final_combine(a0, a1, hm, dinv_col, b, mask_col):
    bm = 512
    return pl.pallas_call(
        _final_combine_body,
        grid=(NP // bm,),
        in_specs=[pl.BlockSpec((bm, H), lambda i: (i, 0)),
                  pl.BlockSpec((bm, H), lambda i: (i, 0)),
                  pl.BlockSpec((bm, H), lambda i: (i, 0)),
                  pl.BlockSpec((bm, 1), lambda i: (i, 0)),
                  pl.BlockSpec((1, H), lambda i: (0, 0)),
                  pl.BlockSpec((bm, 1), lambda i: (i, 0))],
        out_specs=[pl.BlockSpec((1, H), lambda i: (0, 0)),
                   pl.BlockSpec((1, H), lambda i: (0, 0))],
        out_shape=[jax.ShapeDtypeStruct((1, H), jnp.float32),
                   jax.ShapeDtypeStruct((1, H), jnp.float32)],
    )(a0, a1, hm, dinv_col, b[None, :], mask_col)


def _head_body(k1_inv, k2_inv, k3_inv,
               mx1_ref, sm1_ref, mx2_ref, sm2_ref, mx3_ref, sm3_ref,
               wf1a_ref, wf1b_ref, bf1_ref, wf2_ref, bf2_ref, wf3_ref,
               bf3_ref, o_ref):
    ra = (jnp.maximum(mx1_ref[...], 0.0) + jnp.maximum(mx2_ref[...], 0.0)
          + jnp.maximum(mx3_ref[...], 0.0))
    rb = (jnp.maximum(sm1_ref[...] * k1_inv, 0.0)
          + jnp.maximum(sm2_ref[...] * k2_inv, 0.0)
          + jnp.maximum(sm3_ref[...] * k3_inv, 0.0))
    o = jnp.maximum(
        jnp.dot(ra, wf1a_ref[...], preferred_element_type=jnp.float32)
        + jnp.dot(rb, wf1b_ref[...], preferred_element_type=jnp.float32)
        + bf1_ref[...], 0.0)
    o = jnp.maximum(
        jnp.dot(o, wf2_ref[...], preferred_element_type=jnp.float32)
        + bf2_ref[...], 0.0)
    lg = jnp.dot(o, wf3_ref[...], preferred_element_type=jnp.float32) \
        + bf3_ref[...]
    lane = jax.lax.broadcasted_iota(jnp.int32, (1, H), 1)
    lvalid = lane < 2
    m = jnp.max(jnp.where(lvalid, lg, -jnp.inf))
    s = jnp.sum(jnp.where(lvalid, jnp.exp(lg - m), 0.0))
    o_ref[...] = lg - m - jnp.log(s)


def _head(mx1, sm1, mx2, sm2, mx3, sm3, W_fc1, b_fc1, W_fc2, b_fc2,
          W_fc3, b_fc3):
    wf1a = W_fc1[:H]
    wf1b = W_fc1[H:]
    wf2 = jnp.zeros((H, H), jnp.float32).at[:, :H // 2].set(W_fc2)
    bf2 = jnp.zeros((1, H), jnp.float32).at[0, :H // 2].set(b_fc2)
    wf3 = jnp.zeros((H, H), jnp.float32).at[:H // 2, :2].set(W_fc3)
    bf3 = jnp.zeros((1, H), jnp.float32).at[0, :2].set(b_fc3)
    out = pl.pallas_call(
        partial(_head_body, 1.0 / K1, 1.0 / K2, 1.0 / K2),
        in_specs=[pl.BlockSpec((1, H), lambda: (0, 0))] * 6 +
                 [pl.BlockSpec((H, H), lambda: (0, 0)),
                  pl.BlockSpec((H, H), lambda: (0, 0)),
                  pl.BlockSpec((1, H), lambda: (0, 0)),
                  pl.BlockSpec((H, H), lambda: (0, 0)),
                  pl.BlockSpec((1, H), lambda: (0, 0)),
                  pl.BlockSpec((H, H), lambda: (0, 0)),
                  pl.BlockSpec((1, H), lambda: (0, 0))],
        out_specs=pl.BlockSpec((1, H), lambda: (0, 0)),
        out_shape=jax.ShapeDtypeStruct((1, H), jnp.float32),
    )(mx1, sm1, mx2, sm2, mx3, sm3, wf1a, wf1b, b_fc1[None, :], wf2, bf2,
      wf3, bf3)
    return out[:, :2]


def kernel(x, edge_index, batch, W_lin1, b_lin1, W_lin2, b_lin2, W_conv1,
           b_conv1, W_conv2, b_conv2, W_conv3, b_conv3, att1, att2, W_fc1,
           b_fc1, W_fc2, b_fc2, W_fc3, b_fc3):
    xp = jnp.zeros((NP, H), jnp.float32).at[:N0].set(x)
    src = edge_index[0].astype(jnp.int32)
    dst = edge_index[1].astype(jnp.int32)
    pad = jnp.full((EP - E,), DEAD, jnp.int32)
    src2 = jnp.reshape(jnp.concatenate([src, pad]), (ER, 128))
    dst2 = jnp.reshape(jnp.concatenate([dst, pad]), (ER, 128))
    w0 = jnp.reshape(
        jnp.concatenate([jnp.ones((E,), jnp.float32),
                         jnp.zeros((EP - E,), jnp.float32)]), (ER, 128))
    keep0 = (jnp.arange(NP, dtype=jnp.int32) < N0).astype(jnp.float32)[:, None]

    hm1 = _front(xp, W_lin1, b_lin1, W_lin2, b_lin2, W_conv1)
    degp = _sc_deg(dst2)
    gdinv1, pdinv1 = _deg_to_dinv(degp[0][:, None], degp[1][:, None])
    agg = _sc_agg(hm1, src2, dst2, w0, gdinv1[:, 0], 'both')
    h1 = _gcn_combine(agg[0], agg[1], hm1, gdinv1, b_conv1, keep0)

    aggp = _sc_agg(h1, src2, dst2, w0, pdinv1[:, 0], 'dst')
    score1 = _score(h1, aggp[0], aggp[1])
    keep1, scale1, a1v = _topk(score1, keep0, K1)
    xz1 = _scale_rows(h1, scale1)
    ca1, cb1, mx1, sm1 = _att_readout(xz1, att1)
    ee1, den1 = _sc_attn1a(src2, dst2, ca1[:, 0], cb1[:, 0], keep1[:, 0])
    w1 = _sc_attn_norm(ee1, dst2, den1)
    gdinv2, pdinv2 = _den_to_dinv(den1[0][:, None], den1[1][:, None])

    hm2 = _matmul(xz1, W_conv2)
    agg = _sc_agg(hm2, src2, dst2, w1, gdinv2[:, 0], 'both')
    h2 = _gcn_combine(agg[0], agg[1], hm2, gdinv2, b_conv2, keep1)

    aggp = _sc_agg(h2, src2, dst2, w1, pdinv2[:, 0], 'dst')
    score2 = _score(h2, aggp[0], aggp[1])
    keep2, scale2, _ = _topk(score2, keep1, K2)
    xz2 = _scale_rows(h2, scale2)
    ca2, cb2, mx2, sm2 = _att_readout(xz2, att2)
    sh2, dh2, ee2, den2 = _sc_attn2a(src2, dst2, w1, ca2[:, 0], cb2[:, 0],
                                     keep1[:, 0], keep2[:, 0], a1v)
    w2 = _sc_attn_norm(ee2, dh2, den2)
    gdinv3, _ = _den_to_dinv(den2[0][:, None], den2[1][:, None])

    hm3 = _matmul(xz2, W_conv3)
    agg = _sc_agg(hm3, sh2, dh2, w2, gdinv3[:, 0], 'both')
    mx3, sm3 = _final_combine(agg[0], agg[1], hm3, gdinv3, b_conv3, keep2)

    return _head(mx1, sm1, mx2, sm2, mx3, sm3, W_fc1, b_fc1, W_fc2, b_fc2,
                 W_fc3, b_fc3)


# speedup vs baseline: 12.0211x; 3.5210x over previous
from functools import partial

import jax
import jax.numpy as jnp
from jax import lax
from jax.experimental import pallas as pl
from jax.experimental.pallas import tpu as pltpu
from jax.experimental.pallas import tpu_sc as plsc

N0 = 10000
NP = 10240
E = 320000
EP = 327680
ER = EP // 128
AA = 21
H = 128
LAMB = 1.0
K1 = 5000
K2 = 2500
DEAD = NP - 1

_NC, _NS, _L = 2, 16, 16
_NW = _NC * _NS
_ERW = ER // _NW
_RCH = 4
_NCH = _ERW // _RCH
_SLAB = NP // _NS


def _mesh():
    return plsc.VectorSubcoreMesh(core_axis_name="c", subcore_axis_name="s")


def _zero_rows(zref):
    rows, cols = zref.shape
    nv = cols // _L

    def zb(i, _):
        zref[i // nv, pl.ds((i % nv) * _L, _L)] = jnp.zeros((_L,), jnp.float32)
        return 0

    lax.fori_loop(0, rows * nv, zb, 0)


def _zero_vec(zref):
    n = zref.shape[0]

    def zb(i, _):
        zref[pl.ds(i * _L, _L)] = jnp.zeros((_L,), jnp.float32)
        return 0

    lax.fori_loop(0, n // _L, zb, 0)


def _sc_deg(dst2):
    @partial(
        pl.kernel, mesh=_mesh(),
        compiler_params=pltpu.CompilerParams(needs_layout_passes=False),
        out_type=jax.ShapeDtypeStruct((_NC, NP), jnp.float32),
        scratch_types=[
            pltpu.VMEM_SHARED((NP,), jnp.float32),
            pltpu.VMEM((_RCH, 128), jnp.int32),
            pltpu.VMEM((128,), jnp.float32),
            pltpu.VMEM((_SLAB,), jnp.float32),
        ],
    )
    def k(dst_h, out_h, deg_sh, dstv, onesv, zv):
        cid = lax.axis_index("c")
        sid = lax.axis_index("s")
        wid = sid * _NC + cid
        _zero_vec(zv)

        def ob(i, _):
            onesv[pl.ds(i * _L, _L)] = jnp.ones((_L,), jnp.float32)
            return 0

        lax.fori_loop(0, 128 // _L, ob, 0)
        pltpu.sync_copy(zv, deg_sh.at[pl.ds(sid * _SLAB, _SLAB)])
        plsc.subcore_barrier()

        def chunk(g, _):
            erow = wid * _ERW + g * _RCH
            pltpu.sync_copy(dst_h.at[pl.ds(erow, _RCH)], dstv)
            for r in range(_RCH):
                pltpu.sync_copy(onesv, deg_sh.at[dstv.at[r]], add=True)
            return 0

        lax.fori_loop(0, _NCH, chunk, 0)
        plsc.subcore_barrier()
        pltpu.sync_copy(deg_sh.at[pl.ds(sid * _SLAB, _SLAB)],
                        out_h.at[cid, pl.ds(sid * _SLAB, _SLAB)])

    return k(dst2)


def _sc_agg(T, src2, dst2, w2, tab, mode, a1v=None):
    both = mode == "both"
    hoist = a1v is not None
    out_types = [jax.ShapeDtypeStruct((_NC, NP, H), jnp.float32)]
    if hoist:
        out_types.append(jax.ShapeDtypeStruct((_NC, _NS, _L), jnp.float32))

    @partial(
        pl.kernel, mesh=_mesh(),
        compiler_params=pltpu.CompilerParams(needs_layout_passes=False),
        out_type=out_types if hoist else out_types[0],
        scratch_types=[
            pltpu.VMEM_SHARED((NP, H), jnp.float32),
            pltpu.VMEM((NP,), jnp.float32),
            pltpu.VMEM((1, 128), jnp.int32),
            pltpu.VMEM((1, 128), jnp.int32),
            pltpu.VMEM((1, 128), jnp.float32),
            pltpu.VMEM((1, 128), jnp.float32),
            pltpu.VMEM((128, H), jnp.float32),
            pltpu.VMEM((64, H), jnp.float32),
            pltpu.VMEM((_L,), jnp.int32),
            pltpu.VMEM((_L,), jnp.float32),
            pltpu.SemaphoreType.DMA,
        ],
    )
    def k(*refs):
        if hoist:
            (T_h, src_h, dst_h, w_h, tab_h, a1_h, out_h, zs_h,
             acc_sh, tabv, srcv, dstv, wv, coefv, rows, zr, a1m, zv16,
             sem) = refs
        else:
            (T_h, src_h, dst_h, w_h, tab_h, out_h,
             acc_sh, tabv, srcv, dstv, wv, coefv, rows, zr, a1m, zv16,
             sem) = refs
        cid = lax.axis_index("c")
        sid = lax.axis_index("s")
        wid = sid * _NC + cid
        _zero_rows(zr)

        def zslab(i, _):
            pltpu.sync_copy(zr, acc_sh.at[pl.ds(sid * _SLAB + i * 64, 64)])
            return 0

        lax.fori_loop(0, _SLAB // 64, zslab, 0)
        pltpu.sync_copy(tab_h, tabv)
        if hoist:
            pltpu.sync_copy(a1_h, a1m)
        plsc.subcore_barrier()

        def chunk(g, zacc):
            erow = wid * _ERW + g
            pltpu.sync_copy(src_h.at[pl.ds(erow, 1)], srcv)
            pltpu.sync_copy(dst_h.at[pl.ds(erow, 1)], dstv)
            pltpu.sync_copy(w_h.at[pl.ds(erow, 1)], wv)
            if hoist:
                a1l = a1m[...]
            for j in range(128 // _L):
                sl = pl.ds(j * _L, _L)
                d16 = dstv[0, sl]
                s16 = srcv[0, sl]
                c = wv[0, sl] * plsc.load_gather(tabv, [d16])
                if both:
                    c = c * plsc.load_gather(tabv, [s16])
                if hoist:
                    m = (s16 == a1l) & (d16 == a1l)
                    zacc = zacc + jnp.where(m, wv[0, sl], 0.0)
                    spread = ((g % 64) * 128 + j * _L
                              + lax.iota(jnp.int32, _L))
                    srcv[0, sl] = jnp.where(m, spread, s16)
                    dstv[0, sl] = jnp.where(m, spread, d16)
                    c = jnp.where(m, 0.0, c)
                coefv[0, sl] = c
            pltpu.async_copy(T_h.at[srcv.at[0]], rows, sem).wait()

            def ebody(e, _):
                ci = plsc.load_gather(
                    coefv, [jnp.zeros((_L,), jnp.int32),
                            jnp.full((_L,), e, jnp.int32)])
                for j in range(H // _L):
                    sl = pl.ds(j * _L, _L)
                    rows[e, sl] = rows[e, sl] * ci
                return 0

            lax.fori_loop(0, 128, ebody, 0)
            pltpu.sync_copy(rows, acc_sh.at[dstv.at[0]], add=True)
            return zacc

        zacc = lax.fori_loop(0, _ERW, chunk, jnp.zeros((_L,), jnp.float32))
        if hoist:
            zv16[...] = zacc
            pltpu.sync_copy(zv16, zs_h.at[cid, sid])
        plsc.subcore_barrier()

        def drain(i, _):
            row0 = sid * _SLAB + i * 64
            pltpu.sync_copy(acc_sh.at[pl.ds(row0, 64)],
                            out_h.at[cid, pl.ds(row0, 64)])
            return 0

        lax.fori_loop(0, _SLAB // 64, drain, 0)

    if hoist:
        return k(T, src2, dst2, w2, tab, a1v)
    return k(T, src2, dst2, w2, tab)


def _sc_attn1a(src2, dst2, ca, cb, k1t):
    @partial(
        pl.kernel, mesh=_mesh(),
        compiler_params=pltpu.CompilerParams(needs_layout_passes=False),
        out_type=[jax.ShapeDtypeStruct((ER, 128), jnp.float32),
                  jax.ShapeDtypeStruct((_NC, NP), jnp.float32)],
        scratch_types=[
            pltpu.VMEM_SHARED((NP,), jnp.float32),
            pltpu.VMEM((NP,), jnp.float32),
            pltpu.VMEM((NP,), jnp.float32),
            pltpu.VMEM((NP,), jnp.float32),
            pltpu.VMEM((_RCH, 128), jnp.int32),
            pltpu.VMEM((_RCH, 128), jnp.int32),
            pltpu.VMEM((_RCH, 128), jnp.float32),
            pltpu.VMEM((_SLAB,), jnp.float32),
        ],
    )
    def k(src_h, dst_h, ca_h, cb_h, k1_h, ee_h, den_h,
          den_sh, cav, cbv, k1v, srcv, dstv, eev, zv):
        cid = lax.axis_index("c")
        sid = lax.axis_index("s")
        wid = sid * _NC + cid
        _zero_vec(zv)
        pltpu.sync_copy(zv, den_sh.at[pl.ds(sid * _SLAB, _SLAB)])
        pltpu.sync_copy(ca_h, cav)
        pltpu.sync_copy(cb_h, cbv)
        pltpu.sync_copy(k1_h, k1v)
        plsc.subcore_barrier()

        def chunk(g, _):
            erow = wid * _ERW + g * _RCH
            pltpu.sync_copy(src_h.at[pl.ds(erow, _RCH)], srcv)
            pltpu.sync_copy(dst_h.at[pl.ds(erow, _RCH)], dstv)
            for r in range(_RCH):
                for j in range(128 // _L):
                    sl = pl.ds(j * _L, _L)
                    s16 = srcv[r, sl]
                    d16 = dstv[r, sl]
                    kp = ((plsc.load_gather(k1v, [s16]) > 0.5)
                          & (plsc.load_gather(k1v, [d16]) > 0.5))
                    ev = (jnp.maximum(plsc.load_gather(cav, [s16])
                                      + plsc.load_gather(cbv, [d16]), 0.0)
                          + LAMB)
                    eev[r, sl] = jnp.where(kp, jnp.exp(ev), 0.0)
            pltpu.sync_copy(eev, ee_h.at[pl.ds(erow, _RCH)])
            for r in range(_RCH):
                pltpu.sync_copy(eev.at[r], den_sh.at[dstv.at[r]], add=True)
            return 0

        lax.fori_loop(0, _NCH, chunk, 0)
        plsc.subcore_barrier()
        pltpu.sync_copy(den_sh.at[pl.ds(sid * _SLAB, _SLAB)],
                        den_h.at[cid, pl.ds(sid * _SLAB, _SLAB)])

    return k(src2, dst2, ca, cb, k1t)


def _sc_attn_norm(ee2, dst2, den):
    @partial(
        pl.kernel, mesh=_mesh(),
        compiler_params=pltpu.CompilerParams(needs_layout_passes=False),
        out_type=jax.ShapeDtypeStruct((ER, 128), jnp.float32),
        scratch_types=[
            pltpu.VMEM((NP,), jnp.float32),
            pltpu.VMEM((NP,), jnp.float32),
            pltpu.VMEM((_RCH, 128), jnp.int32),
            pltpu.VMEM((_RCH, 128), jnp.float32),
            pltpu.VMEM((_RCH, 128), jnp.float32),
        ],
    )
    def k(ee_h, dst_h, den_h, w_h, denv, den2v, dstv, eev, wv):
        cid = lax.axis_index("c")
        sid = lax.axis_index("s")
        wid = sid * _NC + cid
        pltpu.sync_copy(den_h.at[0], denv)
        pltpu.sync_copy(den_h.at[1], den2v)

        def ab(i, _):
            sl = pl.ds(i * _L, _L)
            denv[sl] = jnp.maximum(denv[sl] + den2v[sl], 1e-16)
            return 0

        lax.fori_loop(0, NP // _L, ab, 0)

        def chunk(g, _):
            erow = wid * _ERW + g * _RCH
            pltpu.sync_copy(dst_h.at[pl.ds(erow, _RCH)], dstv)
            pltpu.sync_copy(ee_h.at[pl.ds(erow, _RCH)], eev)
            for r in range(_RCH):
                for j in range(128 // _L):
                    sl = pl.ds(j * _L, _L)
                    d16 = dstv[r, sl]
                    wv[r, sl] = eev[r, sl] / plsc.load_gather(denv, [d16])
            pltpu.sync_copy(wv, w_h.at[pl.ds(erow, _RCH)])
            return 0

        lax.fori_loop(0, _NCH, chunk, 0)

    return k(ee2, dst2, den)


def _sc_attn2a(src2, dst2, w1, ca, cb, k1t, k2t, a1v):
    @partial(
        pl.kernel, mesh=_mesh(),
        compiler_params=pltpu.CompilerParams(needs_layout_passes=False),
        out_type=[jax.ShapeDtypeStruct((ER, 128), jnp.int32),
                  jax.ShapeDtypeStruct((ER, 128), jnp.int32),
                  jax.ShapeDtypeStruct((ER, 128), jnp.float32),
                  jax.ShapeDtypeStruct((_NC, NP), jnp.float32)],
        scratch_types=[
            pltpu.VMEM_SHARED((NP,), jnp.float32),
            pltpu.VMEM((NP,), jnp.float32),
            pltpu.VMEM((NP,), jnp.float32),
            pltpu.VMEM((NP,), jnp.float32),
            pltpu.VMEM((NP,), jnp.float32),
            pltpu.VMEM((16,), jnp.int32),
            pltpu.VMEM((_RCH, 128), jnp.int32),
            pltpu.VMEM((_RCH, 128), jnp.int32),
            pltpu.VMEM((_RCH, 128), jnp.float32),
            pltpu.VMEM((_RCH, 128), jnp.int32),
            pltpu.VMEM((_RCH, 128), jnp.int32),
            pltpu.VMEM((_RCH, 128), jnp.float32),
            pltpu.VMEM((_SLAB,), jnp.float32),
        ],
    )
    def k(src_h, dst_h, w_h, ca_h, cb_h, k1_h, k2_h, a1_h,
          sh_h, dh_h, ee_h, den_h,
          den_sh, cav, cbv, k1v, k2v, a1vm, srcv, dstv, wv, shv, dhv, eev,
          zv):
        cid = lax.axis_index("c")
        sid = lax.axis_index("s")
        wid = sid * _NC + cid
        _zero_vec(zv)
        pltpu.sync_copy(zv, den_sh.at[pl.ds(sid * _SLAB, _SLAB)])
        pltpu.sync_copy(ca_h, cav)
        pltpu.sync_copy(cb_h, cbv)
        pltpu.sync_copy(k1_h, k1v)
        pltpu.sync_copy(k2_h, k2v)
        pltpu.sync_copy(a1_h, a1vm)
        plsc.subcore_barrier()

        def chunk(g, _):
            erow = wid * _ERW + g * _RCH
            pltpu.sync_copy(src_h.at[pl.ds(erow, _RCH)], srcv)
            pltpu.sync_copy(dst_h.at[pl.ds(erow, _RCH)], dstv)
            pltpu.sync_copy(w_h.at[pl.ds(erow, _RCH)], wv)
            a1l = a1vm[...]
            dead = jnp.full((_L,), DEAD, jnp.int32)
            for r in range(_RCH):
                for j in range(128 // _L):
                    sl = pl.ds(j * _L, _L)
                    s16 = srcv[r, sl]
                    d16 = dstv[r, sl]
                    valid = s16 != dead
                    kp1 = ((plsc.load_gather(k1v, [s16]) > 0.5)
                           & (plsc.load_gather(k1v, [d16]) > 0.5))
                    sh = jnp.where(valid, jnp.where(kp1, s16, a1l), dead)
                    dh = jnp.where(valid, jnp.where(kp1, d16, a1l), dead)
                    kp2 = ((plsc.load_gather(k2v, [sh]) > 0.5)
                           & (plsc.load_gather(k2v, [dh]) > 0.5))
                    wk = jnp.where(kp2, wv[r, sl], 0.0)
                    ev = (jnp.maximum(plsc.load_gather(cav, [sh])
                                      + plsc.load_gather(cbv, [dh]), 0.0)
                          + LAMB * wk)
                    shv[r, sl] = sh
                    dhv[r, sl] = dh
                    eev[r, sl] = jnp.where(kp2, jnp.exp(ev), 0.0)
            pltpu.sync_copy(shv, sh_h.at[pl.ds(erow, _RCH)])
            pltpu.sync_copy(dhv, dh_h.at[pl.ds(erow, _RCH)])
            pltpu.sync_copy(eev, ee_h.at[pl.ds(erow, _RCH)])
            for r in range(_RCH):
                pltpu.sync_copy(eev.at[r], den_sh.at[dhv.at[r]], add=True)
            return 0

        lax.fori_loop(0, _NCH, chunk, 0)
        plsc.subcore_barrier()
        pltpu.sync_copy(den_sh.at[pl.ds(sid * _SLAB, _SLAB)],
                        den_h.at[cid, pl.ds(sid * _SLAB, _SLAB)])

    return k(src2, dst2, w1, ca, cb, k1t, k2t, a1v)


def _front_body(x_ref, w2p_ref, b2p_ref, wc1a_ref, w1p_ref, b1p_ref,
                wc1b_ref, hm_ref):
    x = x_ref[...]
    xb = jnp.maximum(jnp.dot(x, w2p_ref[...],
                             preferred_element_type=jnp.float32)
                     + b2p_ref[...], 0.0)
    xa = jnp.maximum(jnp.dot(x, w1p_ref[...],
                             preferred_element_type=jnp.float32)
                     + b1p_ref[...], 0.0)
    hm_ref[...] = (jnp.dot(xb, wc1a_ref[...], preferred_element_type=jnp.float32)
                   + jnp.dot(xa, wc1b_ref[...], preferred_element_type=jnp.float32))


def _front(x, W_lin1, b_lin1, W_lin2, b_lin2, W_conv1):
    W2p = jnp.zeros((H, H), jnp.float32).at[:AA, :AA].set(W_lin2)
    b2p = jnp.zeros((1, H), jnp.float32).at[0, :AA].set(b_lin2)
    Wc1a = jnp.zeros((H, H), jnp.float32).at[:AA, :].set(W_conv1[:AA])
    W1p = jnp.zeros((H, H), jnp.float32).at[AA:, :].set(W_lin1)
    b1p = jnp.broadcast_to(b_lin1[None, :], (1, H))
    Wc1b = W_conv1[AA:]
    bm = 512
    return pl.pallas_call(
        _front_body,
        grid=(NP // bm,),
        in_specs=[pl.BlockSpec((bm, H), lambda i: (i, 0))] +
                 [pl.BlockSpec((H, H), lambda i: (0, 0)),
                  pl.BlockSpec((1, H), lambda i: (0, 0)),
                  pl.BlockSpec((H, H), lambda i: (0, 0)),
                  pl.BlockSpec((H, H), lambda i: (0, 0)),
                  pl.BlockSpec((1, H), lambda i: (0, 0)),
                  pl.BlockSpec((H, H), lambda i: (0, 0))],
        out_specs=pl.BlockSpec((bm, H), lambda i: (i, 0)),
        out_shape=jax.ShapeDtypeStruct((NP, H), jnp.float32),
    )(x, W2p, b2p, Wc1a, W1p, b1p, Wc1b)


def _matmul_body(x_ref, w_ref, o_ref):
    o_ref[...] = jnp.dot(x_ref[...], w_ref[...],
                         preferred_element_type=jnp.float32)


def _matmul(x, w):
    bm = 512
    return pl.pallas_call(
        _matmul_body,
        grid=(NP // bm,),
        in_specs=[pl.BlockSpec((bm, H), lambda i: (i, 0)),
                  pl.BlockSpec((H, H), lambda i: (0, 0))],
        out_specs=pl.BlockSpec((bm, H), lambda i: (i, 0)),
        out_shape=jax.ShapeDtypeStruct((NP, H), jnp.float32),
    )(x, w)


def _gcn_combine_body(a0_ref, a1_ref, hm_ref, dinv_ref, b_ref, m_ref, o_ref):
    dinv = dinv_ref[...]
    o_ref[...] = m_ref[...] * jnp.maximum(
        a0_ref[...] + a1_ref[...] + dinv * dinv * hm_ref[...] + b_ref[...], 0.0)


def _gcn_combine(a0, a1, hm, dinv_col, b, mask_col):
    bm = 512
    return pl.pallas_call(
        _gcn_combine_body,
        grid=(NP // bm,),
        in_specs=[pl.BlockSpec((bm, H), lambda i: (i, 0)),
                  pl.BlockSpec((bm, H), lambda i: (i, 0)),
                  pl.BlockSpec((bm, H), lambda i: (i, 0)),
                  pl.BlockSpec((bm, 1), lambda i: (i, 0)),
                  pl.BlockSpec((1, H), lambda i: (0, 0)),
                  pl.BlockSpec((bm, 1), lambda i: (i, 0))],
        out_specs=pl.BlockSpec((bm, H), lambda i: (i, 0)),
        out_shape=jax.ShapeDtypeStruct((NP, H), jnp.float32),
    )(a0, a1, hm, dinv_col, b[None, :], mask_col)


def _deg_to_dinv_body(d0_ref, d1_ref, gdinv_ref, pdinv_ref):
    deg = d0_ref[...] + d1_ref[...]
    gdinv_ref[...] = jax.lax.rsqrt(deg + 1.0)
    pdinv_ref[...] = jnp.where(deg > 0, 1.0 / jnp.maximum(deg, 1e-12), 0.0)


def _deg_to_dinv(d0, d1):
    bm = 512
    return pl.pallas_call(
        _deg_to_dinv_body,
        grid=(NP // bm,),
        in_specs=[pl.BlockSpec((bm, 1), lambda i: (i, 0)),
                  pl.BlockSpec((bm, 1), lambda i: (i, 0))],
        out_specs=[pl.BlockSpec((bm, 1), lambda i: (i, 0)),
                   pl.BlockSpec((bm, 1), lambda i: (i, 0))],
        out_shape=[jax.ShapeDtypeStruct((NP, 1), jnp.float32),
                   jax.ShapeDtypeStruct((NP, 1), jnp.float32)],
    )(d0, d1)


def _den_to_dinv_body(d0_ref, d1_ref, gdinv_ref, pdinv_ref):
    deg = ((d0_ref[...] + d1_ref[...]) > 1e-16).astype(jnp.float32)
    gdinv_ref[...] = jax.lax.rsqrt(deg + 1.0)
    pdinv_ref[...] = deg


def _den_to_dinv(d0, d1):
    bm = 512
    return pl.pallas_call(
        _den_to_dinv_body,
        grid=(NP // bm,),
        in_specs=[pl.BlockSpec((bm, 1), lambda i: (i, 0)),
                  pl.BlockSpec((bm, 1), lambda i: (i, 0))],
        out_specs=[pl.BlockSpec((bm, 1), lambda i: (i, 0)),
                   pl.BlockSpec((bm, 1), lambda i: (i, 0))],
        out_shape=[jax.ShapeDtypeStruct((NP, 1), jnp.float32),
                   jax.ShapeDtypeStruct((NP, 1), jnp.float32)],
    )(d0, d1)


def _score_body(x_ref, a0_ref, a1_ref, s_ref):
    s_ref[...] = jnp.sum(jnp.abs(x_ref[...] - a0_ref[...] - a1_ref[...]),
                         axis=1, keepdims=True)


def _score(x, a0, a1):
    bm = 512
    return pl.pallas_call(
        _score_body,
        grid=(NP // bm,),
        in_specs=[pl.BlockSpec((bm, H), lambda i: (i, 0)),
                  pl.BlockSpec((bm, H), lambda i: (i, 0)),
                  pl.BlockSpec((bm, H), lambda i: (i, 0))],
        out_specs=pl.BlockSpec((bm, 1), lambda i: (i, 0)),
        out_shape=jax.ShapeDtypeStruct((NP, 1), jnp.float32),
    )(x, a0, a1)


def _topk_body(k, s_ref, v_ref, keep_ref, scale_ref, amax_ref):
    rows = s_ref.shape[0]
    score = s_ref[...]
    valid = v_ref[...] > 0.5
    ridx = jax.lax.broadcasted_iota(jnp.int32, (rows, H), 0)
    cidx = jax.lax.broadcasted_iota(jnp.int32, (rows, H), 1)
    flat = ridx * H + cidx
    bits = jax.lax.bitcast_convert_type(score, jnp.uint32)
    keys = jnp.where(bits >> 31 != 0, ~bits, bits | jnp.uint32(0x80000000))
    keys = jnp.where(valid, keys, jnp.uint32(0))
    sgn = jnp.uint32(0x80000000)
    ik = jax.lax.bitcast_convert_type(keys ^ sgn, jnp.int32)

    def body(i, carry):
        prefix, kk = carry
        bit = jnp.uint32(1) << (31 - i)
        cand = prefix | bit
        mask = ~(bit - jnp.uint32(1))
        icand = jax.lax.bitcast_convert_type(cand ^ sgn, jnp.int32)
        imasked = jax.lax.bitcast_convert_type((keys & mask) ^ sgn, jnp.int32)
        cnt = jnp.sum((imasked >= icand).astype(jnp.int32))
        take = cnt >= kk
        return (jnp.where(take, cand, prefix),
                jnp.where(take, kk, kk - cnt))

    thr, _ = jax.lax.fori_loop(0, 32, body, (jnp.uint32(0), jnp.int32(k)))
    ithr = jax.lax.bitcast_convert_type(thr ^ sgn, jnp.int32)
    gt = ik > ithr
    eq = valid & (ik == ithr)
    n_gt = jnp.sum(gt.astype(jnp.int32))

    ut = (jax.lax.broadcasted_iota(jnp.int32, (H, H), 0)
          <= jax.lax.broadcasted_iota(jnp.int32, (H, H), 1)).astype(jnp.float32)
    lt_strict = (jax.lax.broadcasted_iota(jnp.int32, (rows, rows), 1)
                 < jax.lax.broadcasted_iota(jnp.int32, (rows, rows), 0)
                 ).astype(jnp.float32)

    def cumsum2d(m):
        mf = m.astype(jnp.float32)
        within = jnp.dot(mf, ut, preferred_element_type=jnp.float32)
        row_tot = jnp.sum(mf, axis=1, keepdims=True)
        excl = jnp.dot(lt_strict, jnp.broadcast_to(row_tot, (rows, H)),
                       preferred_element_type=jnp.float32)
        return (within + excl).astype(jnp.int32)

    eq_rank = cumsum2d(eq) - 1
    keep = gt | (eq & (eq_rank < (k - n_gt)))
    maxk = jnp.max(ik)
    amax = jnp.min(jnp.where(ik == maxk, flat, jnp.int32(2**30)))
    keep_ref[...] = keep.astype(jnp.float32)
    scale_ref[...] = jnp.where(keep, jnp.tanh(score), 0.0)
    amax_ref[...] = jnp.full((1, H), amax, jnp.int32)


def _topk(score_col, valid_col, k):
    rows = NP // H
    s_rs = jnp.reshape(score_col[:, 0], (rows, H))
    v_rs = jnp.reshape(valid_col[:, 0], (rows, H))
    keep_rs, scale_rs, amax_o = pl.pallas_call(
        partial(_topk_body, k),
        in_specs=[pl.BlockSpec((rows, H), lambda: (0, 0)),
                  pl.BlockSpec((rows, H), lambda: (0, 0))],
        out_specs=[pl.BlockSpec((rows, H), lambda: (0, 0)),
                   pl.BlockSpec((rows, H), lambda: (0, 0)),
                   pl.BlockSpec((1, H), lambda: (0, 0))],
        out_shape=[jax.ShapeDtypeStruct((rows, H), jnp.float32),
                   jax.ShapeDtypeStruct((rows, H), jnp.float32),
                   jax.ShapeDtypeStruct((1, H), jnp.int32)],
    )(s_rs, v_rs)
    keep = jnp.reshape(keep_rs, (NP, 1))
    scale = jnp.reshape(scale_rs, (NP, 1))
    return keep, scale, amax_o[0, :16]


def _scale_rows_body(x_ref, t_ref, o_ref):
    o_ref[...] = x_ref[...] * t_ref[...]


def _scale_rows(x, t_col):
    bm = 512
    return pl.pallas_call(
        _scale_rows_body,
        grid=(NP // bm,),
        in_specs=[pl.BlockSpec((bm, H), lambda i: (i, 0)),
                  pl.BlockSpec((bm, 1), lambda i: (i, 0))],
        out_specs=pl.BlockSpec((bm, H), lambda i: (i, 0)),
        out_shape=jax.ShapeDtypeStruct((NP, H), jnp.float32),
    )(x, t_col)


def _att_readout_body(xk_ref, atta_ref, attb_ref, ca_ref, cb_ref, mx_ref,
                      sm_ref):
    xk = xk_ref[...]
    ca_ref[...] = jnp.sum(xk * atta_ref[...], axis=1, keepdims=True)
    cb_ref[...] = jnp.sum(xk * attb_ref[...], axis=1, keepdims=True)
    i = pl.program_id(0)
    bmax = jnp.max(xk, axis=0, keepdims=True)
    bsum = jnp.sum(xk, axis=0, keepdims=True)

    @pl.when(i == 0)
    def _():
        mx_ref[...] = bmax
        sm_ref[...] = bsum

    @pl.when(i != 0)
    def _():
        mx_ref[...] = jnp.maximum(mx_ref[...], bmax)
        sm_ref[...] = sm_ref[...] + bsum


def _att_readout(xk, att):
    bm = 512
    atta = att[None, :H]
    attb = att[None, H:]
    return pl.pallas_call(
        _att_readout_body,
        grid=(NP // bm,),
        in_specs=[pl.BlockSpec((bm, H), lambda i: (i, 0)),
                  pl.BlockSpec((1, H), lambda i: (0, 0)),
                  pl.BlockSpec((1, H), lambda i: (0, 0))],
        out_specs=[pl.BlockSpec((bm, 1), lambda i: (i, 0)),
                   pl.BlockSpec((bm, 1), lambda i: (i, 0)),
                   pl.BlockSpec((1, H), lambda i: (0, 0)),
                   pl.BlockSpec((1, H), lambda i: (0, 0))],
        out_shape=[jax.ShapeDtypeStruct((NP, 1), jnp.float32),
                   jax.ShapeDtypeStruct((NP, 1), jnp.float32),
                   jax.ShapeDtypeStruct((1, H), jnp.float32),
                   jax.ShapeDtypeStruct((1, H), jnp.float32)],
    )(xk, atta, attb)


def _final_combine_body(a0_ref, a1_ref, hm_ref, dinv_ref, b_ref, m_ref,
                        zs_ref, an_ref, mx_ref, sm_ref):
    dinv = dinv_ref[...]
    i = pl.program_id(0)
    bm = hm_ref.shape[0]
    zw = jnp.sum(zs_ref[...])
    a1s = an_ref[0, 0]
    rowid = i * bm + jax.lax.broadcasted_iota(jnp.int32, (bm, H), 0)
    factor = jnp.where(rowid == a1s, 1.0 + zw, 1.0)
    h = m_ref[...] * jnp.maximum(
        a0_ref[...] + a1_ref[...] + dinv * dinv * hm_ref[...] * factor
        + b_ref[...], 0.0)
    bmax = jnp.max(h, axis=0, keepdims=True)
    bsum = jnp.sum(h, axis=0, keepdims=True)

    @pl.when(i == 0)
    def _():
        mx_ref[...] = bmax
        sm_ref[...] = bsum

    @pl.when(i != 0)
    def _():
        mx_ref[...] = jnp.maximum(mx_ref[...], bmax)
        sm_ref[...] = sm_ref[...] + bsum


def _final_combine(a0, a1, hm, dinv_col, b, mask_col, zs_rs, a1_bc):
    bm = 512
    return pl.pallas_call(
        _final_combine_body,
        grid=(NP // bm,),
        in_specs=[pl.BlockSpec((bm, H), lambda i: (i, 0)),
                  pl.BlockSpec((bm, H), lambda i: (i, 0)),
                  pl.BlockSpec((bm, H), lambda i: (i, 0)),
                  pl.BlockSpec((bm, 1), lambda i: (i, 0)),
                  pl.BlockSpec((1, H), lambda i: (0, 0)),
                  pl.BlockSpec((bm, 1), lambda i: (i, 0)),
                  pl.BlockSpec((4, H), lambda i: (0, 0)),
                  pl.BlockSpec((1, H), lambda i: (0, 0))],
        out_specs=[pl.BlockSpec((1, H), lambda i: (0, 0)),
                   pl.BlockSpec((1, H), lambda i: (0, 0))],
        out_shape=[jax.ShapeDtypeStruct((1, H), jnp.float32),
                   jax.ShapeDtypeStruct((1, H), jnp.float32)],
    )(a0, a1, hm, dinv_col, b[None, :], mask_col, zs_rs, a1_bc)


def _head_body(k1_inv, k2_inv, k3_inv,
               mx1_ref, sm1_ref, mx2_ref, sm2_ref, mx3_ref, sm3_ref,
               wf1a_ref, wf1b_ref, bf1_ref, wf2_ref, bf2_ref, wf3_ref,
               bf3_ref, o_ref):
    ra = (jnp.maximum(mx1_ref[...], 0.0) + jnp.maximum(mx2_ref[...], 0.0)
          + jnp.maximum(mx3_ref[...], 0.0))
    rb = (jnp.maximum(sm1_ref[...] * k1_inv, 0.0)
          + jnp.maximum(sm2_ref[...] * k2_inv, 0.0)
          + jnp.maximum(sm3_ref[...] * k3_inv, 0.0))
    o = jnp.maximum(
        jnp.dot(ra, wf1a_ref[...], preferred_element_type=jnp.float32)
        + jnp.dot(rb, wf1b_ref[...], preferred_element_type=jnp.float32)
        + bf1_ref[...], 0.0)
    o = jnp.maximum(
        jnp.dot(o, wf2_ref[...], preferred_element_type=jnp.float32)
        + bf2_ref[...], 0.0)
    lg = jnp.dot(o, wf3_ref[...], preferred_element_type=jnp.float32) \
        + bf3_ref[...]
    lane = jax.lax.broadcasted_iota(jnp.int32, (1, H), 1)
    lvalid = lane < 2
    m = jnp.max(jnp.where(lvalid, lg, -jnp.inf))
    s = jnp.sum(jnp.where(lvalid, jnp.exp(lg - m), 0.0))
    o_ref[...] = lg - m - jnp.log(s)


def _head(mx1, sm1, mx2, sm2, mx3, sm3, W_fc1, b_fc1, W_fc2, b_fc2,
          W_fc3, b_fc3):
    wf1a = W_fc1[:H]
    wf1b = W_fc1[H:]
    wf2 = jnp.zeros((H, H), jnp.float32).at[:, :H // 2].set(W_fc2)
    bf2 = jnp.zeros((1, H), jnp.float32).at[0, :H // 2].set(b_fc2)
    wf3 = jnp.zeros((H, H), jnp.float32).at[:H // 2, :2].set(W_fc3)
    bf3 = jnp.zeros((1, H), jnp.float32).at[0, :2].set(b_fc3)
    out = pl.pallas_call(
        partial(_head_body, 1.0 / K1, 1.0 / K2, 1.0 / K2),
        in_specs=[pl.BlockSpec((1, H), lambda: (0, 0))] * 6 +
                 [pl.BlockSpec((H, H), lambda: (0, 0)),
                  pl.BlockSpec((H, H), lambda: (0, 0)),
                  pl.BlockSpec((1, H), lambda: (0, 0)),
                  pl.BlockSpec((H, H), lambda: (0, 0)),
                  pl.BlockSpec((1, H), lambda: (0, 0)),
                  pl.BlockSpec((H, H), lambda: (0, 0)),
                  pl.BlockSpec((1, H), lambda: (0, 0))],
        out_specs=pl.BlockSpec((1, H), lambda: (0, 0)),
        out_shape=jax.ShapeDtypeStruct((1, H), jnp.float32),
    )(mx1, sm1, mx2, sm2, mx3, sm3, wf1a, wf1b, b_fc1[None, :], wf2, bf2,
      wf3, bf3)
    return out[:, :2]


def kernel(x, edge_index, batch, W_lin1, b_lin1, W_lin2, b_lin2, W_conv1,
           b_conv1, W_conv2, b_conv2, W_conv3, b_conv3, att1, att2, W_fc1,
           b_fc1, W_fc2, b_fc2, W_fc3, b_fc3):
    xp = jnp.zeros((NP, H), jnp.float32).at[:N0].set(x)
    src = edge_index[0].astype(jnp.int32)
    dst = edge_index[1].astype(jnp.int32)
    pad = jnp.full((EP - E,), DEAD, jnp.int32)
    src2 = jnp.reshape(jnp.concatenate([src, pad]), (ER, 128))
    dst2 = jnp.reshape(jnp.concatenate([dst, pad]), (ER, 128))
    w0 = jnp.reshape(
        jnp.concatenate([jnp.ones((E,), jnp.float32),
                         jnp.zeros((EP - E,), jnp.float32)]), (ER, 128))
    keep0 = (jnp.arange(NP, dtype=jnp.int32) < N0).astype(jnp.float32)[:, None]

    hm1 = _front(xp, W_lin1, b_lin1, W_lin2, b_lin2, W_conv1)
    degp = _sc_deg(dst2)
    gdinv1, pdinv1 = _deg_to_dinv(degp[0][:, None], degp[1][:, None])
    agg = _sc_agg(hm1, src2, dst2, w0, gdinv1[:, 0], 'both')
    h1 = _gcn_combine(agg[0], agg[1], hm1, gdinv1, b_conv1, keep0)

    aggp = _sc_agg(h1, src2, dst2, w0, pdinv1[:, 0], 'dst')
    score1 = _score(h1, aggp[0], aggp[1])
    keep1, scale1, a1v = _topk(score1, keep0, K1)
    xz1 = _scale_rows(h1, scale1)
    ca1, cb1, mx1, sm1 = _att_readout(xz1, att1)
    ee1, den1 = _sc_attn1a(src2, dst2, ca1[:, 0], cb1[:, 0], keep1[:, 0])
    w1 = _sc_attn_norm(ee1, dst2, den1)
    gdinv2, pdinv2 = _den_to_dinv(den1[0][:, None], den1[1][:, None])

    hm2 = _matmul(xz1, W_conv2)
    agg = _sc_agg(hm2, src2, dst2, w1, gdinv2[:, 0], 'both')
    h2 = _gcn_combine(agg[0], agg[1], hm2, gdinv2, b_conv2, keep1)

    aggp = _sc_agg(h2, src2, dst2, w1, pdinv2[:, 0], 'dst')
    score2 = _score(h2, aggp[0], aggp[1])
    keep2, scale2, _ = _topk(score2, keep1, K2)
    xz2 = _scale_rows(h2, scale2)
    ca2, cb2, mx2, sm2 = _att_readout(xz2, att2)
    sh2, dh2, ee2, den2 = _sc_attn2a(src2, dst2, w1, ca2[:, 0], cb2[:, 0],
                                     keep1[:, 0], keep2[:, 0], a1v)
    w2 = _sc_attn_norm(ee2, dh2, den2)
    gdinv3, _ = _den_to_dinv(den2[0][:, None], den2[1][:, None])

    hm3 = _matmul(xz2, W_conv3)
    agg, zs = _sc_agg(hm3, sh2, dh2, w2, gdinv3[:, 0], 'both', a1v)
    zs_rs = jnp.reshape(zs, (4, H))
    a1_bc = jnp.broadcast_to(a1v[:1][None, :], (1, H))
    mx3, sm3 = _final_combine(agg[0], agg[1], hm3, gdinv3, b_conv3, keep2,
                              zs_rs, a1_bc)

    return _head(mx1, sm1, mx2, sm2, mx3, sm3, W_fc1, b_fc1, W_fc2, b_fc2,
                 W_fc3, b_fc3)


# speedup vs baseline: 16.6383x; 1.3841x over previous
from functools import partial

import jax
import jax.numpy as jnp
from jax import lax
from jax.experimental import pallas as pl
from jax.experimental.pallas import tpu as pltpu
from jax.experimental.pallas import tpu_sc as plsc

N0 = 10000
NP = 10240
E = 320000
EP = 327680
ER = EP // 128
AA = 21
H = 128
LAMB = 1.0
K1 = 5000
K2 = 2500
DEAD = NP - 1

_NC, _NS, _L = 2, 16, 16
_NW = _NC * _NS
_ERW = ER // _NW
_RCH = 4
_NCH = _ERW // _RCH
_SLAB = NP // _NS


def _mesh():
    return plsc.VectorSubcoreMesh(core_axis_name="c", subcore_axis_name="s")


def _zero_rows(zref):
    rows, cols = zref.shape
    nv = cols // _L

    def zb(i, _):
        zref[i // nv, pl.ds((i % nv) * _L, _L)] = jnp.zeros((_L,), jnp.float32)
        return 0

    lax.fori_loop(0, rows * nv, zb, 0)


def _zero_vec(zref):
    n = zref.shape[0]

    def zb(i, _):
        zref[pl.ds(i * _L, _L)] = jnp.zeros((_L,), jnp.float32)
        return 0

    lax.fori_loop(0, n // _L, zb, 0)


def _sc_deg(dst2):
    @partial(
        pl.kernel, mesh=_mesh(),
        compiler_params=pltpu.CompilerParams(needs_layout_passes=False),
        out_type=jax.ShapeDtypeStruct((_NC, NP), jnp.float32),
        scratch_types=[
            pltpu.VMEM_SHARED((NP,), jnp.float32),
            pltpu.VMEM((_RCH, 128), jnp.int32),
            pltpu.VMEM((128,), jnp.float32),
            pltpu.VMEM((_SLAB,), jnp.float32),
        ],
    )
    def k(dst_h, out_h, deg_sh, dstv, onesv, zv):
        cid = lax.axis_index("c")
        sid = lax.axis_index("s")
        wid = sid * _NC + cid
        _zero_vec(zv)

        def ob(i, _):
            onesv[pl.ds(i * _L, _L)] = jnp.ones((_L,), jnp.float32)
            return 0

        lax.fori_loop(0, 128 // _L, ob, 0)
        pltpu.sync_copy(zv, deg_sh.at[pl.ds(sid * _SLAB, _SLAB)])
        plsc.subcore_barrier()

        def chunk(g, _):
            erow = wid * _ERW + g * _RCH
            pltpu.sync_copy(dst_h.at[pl.ds(erow, _RCH)], dstv)
            for r in range(_RCH):
                pltpu.sync_copy(onesv, deg_sh.at[dstv.at[r]], add=True)
            return 0

        lax.fori_loop(0, _NCH, chunk, 0)
        plsc.subcore_barrier()
        pltpu.sync_copy(deg_sh.at[pl.ds(sid * _SLAB, _SLAB)],
                        out_h.at[cid, pl.ds(sid * _SLAB, _SLAB)])

    return k(dst2)


def _sc_agg(T, src2, dst2, w2, tab, mode, a1v=None):
    both = mode == "both"
    hoist = a1v is not None
    out_types = [jax.ShapeDtypeStruct((_NC, NP, H), jnp.float32)]
    if hoist:
        out_types.append(jax.ShapeDtypeStruct((_NC, _NS, _L), jnp.float32))

    @partial(
        pl.kernel, mesh=_mesh(),
        compiler_params=pltpu.CompilerParams(needs_layout_passes=False),
        out_type=out_types if hoist else out_types[0],
        scratch_types=[
            pltpu.VMEM_SHARED((NP, H), jnp.float32),
            pltpu.VMEM((NP,), jnp.float32),
            pltpu.VMEM((2, 128), jnp.int32),
            pltpu.VMEM((2, 128), jnp.int32),
            pltpu.VMEM((2, 128), jnp.float32),
            pltpu.VMEM((2, 128), jnp.float32),
            pltpu.VMEM((128, H), jnp.float32),
            pltpu.VMEM((128, H), jnp.float32),
            pltpu.VMEM((16, H), jnp.float32),
            pltpu.VMEM((_L,), jnp.int32),
            pltpu.VMEM((_L,), jnp.float32),
            pltpu.SemaphoreType.DMA,
            pltpu.SemaphoreType.DMA,
        ],
    )
    def k(*refs):
        if hoist:
            (T_h, src_h, dst_h, w_h, tab_h, a1_h, out_h, zs_h,
             acc_sh, tabv, srcv, dstv, wv, coefv, rows_a, rows_b, zr, a1m,
             zv16, sem_a, sem_b) = refs
        else:
            (T_h, src_h, dst_h, w_h, tab_h, out_h,
             acc_sh, tabv, srcv, dstv, wv, coefv, rows_a, rows_b, zr, a1m,
             zv16, sem_a, sem_b) = refs
        cid = lax.axis_index("c")
        sid = lax.axis_index("s")
        wid = sid * _NC + cid
        _zero_rows(zr)

        def zslab(i, _):
            pltpu.sync_copy(zr, acc_sh.at[pl.ds(sid * _SLAB + i * 16, 16)])
            return 0

        lax.fori_loop(0, _SLAB // 16, zslab, 0)
        pltpu.sync_copy(tab_h, tabv)
        if hoist:
            pltpu.sync_copy(a1_h, a1m)
        plsc.subcore_barrier()

        def prep(g, b, zacc):
            erow = wid * _ERW + g
            pltpu.sync_copy(src_h.at[pl.ds(erow, 1)], srcv.at[pl.ds(b, 1)])
            pltpu.sync_copy(dst_h.at[pl.ds(erow, 1)], dstv.at[pl.ds(b, 1)])
            pltpu.sync_copy(w_h.at[pl.ds(erow, 1)], wv.at[pl.ds(b, 1)])
            if hoist:
                a1l = a1m[...]
            for j in range(128 // _L):
                sl = pl.ds(j * _L, _L)
                d16 = dstv[b, sl]
                s16 = srcv[b, sl]
                c = wv[b, sl] * plsc.load_gather(tabv, [d16])
                if both:
                    c = c * plsc.load_gather(tabv, [s16])
                if hoist:
                    m = (s16 == a1l) & (d16 == a1l)
                    zacc = zacc + jnp.where(m, wv[b, sl], 0.0)
                    spread = ((g % 64) * 128 + j * _L
                              + lax.iota(jnp.int32, _L))
                    srcv[b, sl] = jnp.where(m, spread, s16)
                    dstv[b, sl] = jnp.where(m, spread, d16)
                    c = jnp.where(m, 0.0, c)
                coefv[b, sl] = c
            rbuf = rows_a if b == 0 else rows_b
            gsem = sem_a if b == 0 else sem_b
            pltpu.async_copy(T_h.at[srcv.at[b]], rbuf, gsem)
            return zacc

        def work(b):
            rbuf = rows_a if b == 0 else rows_b
            gsem = sem_a if b == 0 else sem_b
            pltpu.make_async_copy(T_h.at[srcv.at[b]], rbuf, gsem).wait()

            def ebody(e, _):
                ci = plsc.load_gather(
                    coefv, [jnp.full((_L,), b, jnp.int32),
                            jnp.full((_L,), e, jnp.int32)])
                for j in range(H // _L):
                    sl = pl.ds(j * _L, _L)
                    rbuf[e, sl] = rbuf[e, sl] * ci
                return 0

            lax.fori_loop(0, 128, ebody, 0)
            pltpu.sync_copy(rbuf, acc_sh.at[dstv.at[b]], add=True)

        zacc0 = prep(wid * 0, 0, jnp.zeros((_L,), jnp.float32))

        def chunk2(g2, zacc):
            g0 = 2 * g2
            zacc = lax.cond(g0 + 1 < _ERW,
                            lambda z: prep(g0 + 1, 1, z), lambda z: z, zacc)
            work(0)
            zacc = lax.cond(g0 + 2 < _ERW,
                            lambda z: prep(g0 + 2, 0, z), lambda z: z, zacc)

            @pl.when(g0 + 1 < _ERW)
            def _():
                work(1)
            return zacc

        zacc = lax.fori_loop(0, (_ERW + 1) // 2, chunk2, zacc0)
        if hoist:
            zv16[...] = zacc
            pltpu.sync_copy(zv16, zs_h.at[cid, sid])
        plsc.subcore_barrier()

        def drain(i, _):
            row0 = sid * _SLAB + i * 64
            pltpu.sync_copy(acc_sh.at[pl.ds(row0, 64)],
                            out_h.at[cid, pl.ds(row0, 64)])
            return 0

        lax.fori_loop(0, _SLAB // 64, drain, 0)

    if hoist:
        return k(T, src2, dst2, w2, tab, a1v)
    return k(T, src2, dst2, w2, tab)


def _sc_attn1a(src2, dst2, ca, cb, k1t):
    @partial(
        pl.kernel, mesh=_mesh(),
        compiler_params=pltpu.CompilerParams(needs_layout_passes=False),
        out_type=[jax.ShapeDtypeStruct((ER, 128), jnp.float32),
                  jax.ShapeDtypeStruct((_NC, NP), jnp.float32)],
        scratch_types=[
            pltpu.VMEM_SHARED((NP,), jnp.float32),
            pltpu.VMEM((NP,), jnp.float32),
            pltpu.VMEM((NP,), jnp.float32),
            pltpu.VMEM((NP,), jnp.float32),
            pltpu.VMEM((_RCH, 128), jnp.int32),
            pltpu.VMEM((_RCH, 128), jnp.int32),
            pltpu.VMEM((_RCH, 128), jnp.float32),
            pltpu.VMEM((_SLAB,), jnp.float32),
        ],
    )
    def k(src_h, dst_h, ca_h, cb_h, k1_h, ee_h, den_h,
          den_sh, cav, cbv, k1v, srcv, dstv, eev, zv):
        cid = lax.axis_index("c")
        sid = lax.axis_index("s")
        wid = sid * _NC + cid
        _zero_vec(zv)
        pltpu.sync_copy(zv, den_sh.at[pl.ds(sid * _SLAB, _SLAB)])
        pltpu.sync_copy(ca_h, cav)
        pltpu.sync_copy(cb_h, cbv)
        pltpu.sync_copy(k1_h, k1v)
        plsc.subcore_barrier()

        def chunk(g, _):
            erow = wid * _ERW + g * _RCH
            pltpu.sync_copy(src_h.at[pl.ds(erow, _RCH)], srcv)
            pltpu.sync_copy(dst_h.at[pl.ds(erow, _RCH)], dstv)
            for r in range(_RCH):
                for j in range(128 // _L):
                    sl = pl.ds(j * _L, _L)
                    s16 = srcv[r, sl]
                    d16 = dstv[r, sl]
                    kp = ((plsc.load_gather(k1v, [s16]) > 0.5)
                          & (plsc.load_gather(k1v, [d16]) > 0.5))
                    ev = (jnp.maximum(plsc.load_gather(cav, [s16])
                                      + plsc.load_gather(cbv, [d16]), 0.0)
                          + LAMB)
                    eev[r, sl] = jnp.where(kp, jnp.exp(ev), 0.0)
            pltpu.sync_copy(eev, ee_h.at[pl.ds(erow, _RCH)])
            for r in range(_RCH):
                pltpu.sync_copy(eev.at[r], den_sh.at[dstv.at[r]], add=True)
            return 0

        lax.fori_loop(0, _NCH, chunk, 0)
        plsc.subcore_barrier()
        pltpu.sync_copy(den_sh.at[pl.ds(sid * _SLAB, _SLAB)],
                        den_h.at[cid, pl.ds(sid * _SLAB, _SLAB)])

    return k(src2, dst2, ca, cb, k1t)


def _sc_attn_norm(ee2, dst2, den):
    @partial(
        pl.kernel, mesh=_mesh(),
        compiler_params=pltpu.CompilerParams(needs_layout_passes=False),
        out_type=jax.ShapeDtypeStruct((ER, 128), jnp.float32),
        scratch_types=[
            pltpu.VMEM((NP,), jnp.float32),
            pltpu.VMEM((NP,), jnp.float32),
            pltpu.VMEM((_RCH, 128), jnp.int32),
            pltpu.VMEM((_RCH, 128), jnp.float32),
            pltpu.VMEM((_RCH, 128), jnp.float32),
        ],
    )
    def k(ee_h, dst_h, den_h, w_h, denv, den2v, dstv, eev, wv):
        cid = lax.axis_index("c")
        sid = lax.axis_index("s")
        wid = sid * _NC + cid
        pltpu.sync_copy(den_h.at[0], denv)
        pltpu.sync_copy(den_h.at[1], den2v)

        def ab(i, _):
            sl = pl.ds(i * _L, _L)
            denv[sl] = jnp.maximum(denv[sl] + den2v[sl], 1e-16)
            return 0

        lax.fori_loop(0, NP // _L, ab, 0)

        def chunk(g, _):
            erow = wid * _ERW + g * _RCH
            pltpu.sync_copy(dst_h.at[pl.ds(erow, _RCH)], dstv)
            pltpu.sync_copy(ee_h.at[pl.ds(erow, _RCH)], eev)
            for r in range(_RCH):
                for j in range(128 // _L):
                    sl = pl.ds(j * _L, _L)
                    d16 = dstv[r, sl]
                    wv[r, sl] = eev[r, sl] / plsc.load_gather(denv, [d16])
            pltpu.sync_copy(wv, w_h.at[pl.ds(erow, _RCH)])
            return 0

        lax.fori_loop(0, _NCH, chunk, 0)

    return k(ee2, dst2, den)


def _sc_attn2a(src2, dst2, w1, ca, cb, k1t, k2t, a1v):
    @partial(
        pl.kernel, mesh=_mesh(),
        compiler_params=pltpu.CompilerParams(needs_layout_passes=False),
        out_type=[jax.ShapeDtypeStruct((ER, 128), jnp.int32),
                  jax.ShapeDtypeStruct((ER, 128), jnp.int32),
                  jax.ShapeDtypeStruct((ER, 128), jnp.float32),
                  jax.ShapeDtypeStruct((_NC, NP), jnp.float32)],
        scratch_types=[
            pltpu.VMEM_SHARED((NP,), jnp.float32),
            pltpu.VMEM((NP,), jnp.float32),
            pltpu.VMEM((NP,), jnp.float32),
            pltpu.VMEM((NP,), jnp.float32),
            pltpu.VMEM((NP,), jnp.float32),
            pltpu.VMEM((16,), jnp.int32),
            pltpu.VMEM((_RCH, 128), jnp.int32),
            pltpu.VMEM((_RCH, 128), jnp.int32),
            pltpu.VMEM((_RCH, 128), jnp.float32),
            pltpu.VMEM((_RCH, 128), jnp.int32),
            pltpu.VMEM((_RCH, 128), jnp.int32),
            pltpu.VMEM((_RCH, 128), jnp.float32),
            pltpu.VMEM((_SLAB,), jnp.float32),
        ],
    )
    def k(src_h, dst_h, w_h, ca_h, cb_h, k1_h, k2_h, a1_h,
          sh_h, dh_h, ee_h, den_h,
          den_sh, cav, cbv, k1v, k2v, a1vm, srcv, dstv, wv, shv, dhv, eev,
          zv):
        cid = lax.axis_index("c")
        sid = lax.axis_index("s")
        wid = sid * _NC + cid
        _zero_vec(zv)
        pltpu.sync_copy(zv, den_sh.at[pl.ds(sid * _SLAB, _SLAB)])
        pltpu.sync_copy(ca_h, cav)
        pltpu.sync_copy(cb_h, cbv)
        pltpu.sync_copy(k1_h, k1v)
        pltpu.sync_copy(k2_h, k2v)
        pltpu.sync_copy(a1_h, a1vm)
        plsc.subcore_barrier()

        def chunk(g, _):
            erow = wid * _ERW + g * _RCH
            pltpu.sync_copy(src_h.at[pl.ds(erow, _RCH)], srcv)
            pltpu.sync_copy(dst_h.at[pl.ds(erow, _RCH)], dstv)
            pltpu.sync_copy(w_h.at[pl.ds(erow, _RCH)], wv)
            a1l = a1vm[...]
            dead = jnp.full((_L,), DEAD, jnp.int32)
            for r in range(_RCH):
                for j in range(128 // _L):
                    sl = pl.ds(j * _L, _L)
                    s16 = srcv[r, sl]
                    d16 = dstv[r, sl]
                    valid = s16 != dead
                    kp1 = ((plsc.load_gather(k1v, [s16]) > 0.5)
                           & (plsc.load_gather(k1v, [d16]) > 0.5))
                    sh = jnp.where(valid, jnp.where(kp1, s16, a1l), dead)
                    dh = jnp.where(valid, jnp.where(kp1, d16, a1l), dead)
                    kp2 = ((plsc.load_gather(k2v, [sh]) > 0.5)
                           & (plsc.load_gather(k2v, [dh]) > 0.5))
                    wk = jnp.where(kp2, wv[r, sl], 0.0)
                    ev = (jnp.maximum(plsc.load_gather(cav, [sh])
                                      + plsc.load_gather(cbv, [dh]), 0.0)
                          + LAMB * wk)
                    shv[r, sl] = sh
                    dhv[r, sl] = dh
                    eev[r, sl] = jnp.where(kp2, jnp.exp(ev), 0.0)
            pltpu.sync_copy(shv, sh_h.at[pl.ds(erow, _RCH)])
            pltpu.sync_copy(dhv, dh_h.at[pl.ds(erow, _RCH)])
            pltpu.sync_copy(eev, ee_h.at[pl.ds(erow, _RCH)])
            for r in range(_RCH):
                pltpu.sync_copy(eev.at[r], den_sh.at[dhv.at[r]], add=True)
            return 0

        lax.fori_loop(0, _NCH, chunk, 0)
        plsc.subcore_barrier()
        pltpu.sync_copy(den_sh.at[pl.ds(sid * _SLAB, _SLAB)],
                        den_h.at[cid, pl.ds(sid * _SLAB, _SLAB)])

    return k(src2, dst2, w1, ca, cb, k1t, k2t, a1v)


def _front_body(x_ref, w2p_ref, b2p_ref, wc1a_ref, w1p_ref, b1p_ref,
                wc1b_ref, hm_ref):
    x = x_ref[...]
    xb = jnp.maximum(jnp.dot(x, w2p_ref[...],
                             preferred_element_type=jnp.float32)
                     + b2p_ref[...], 0.0)
    xa = jnp.maximum(jnp.dot(x, w1p_ref[...],
                             preferred_element_type=jnp.float32)
                     + b1p_ref[...], 0.0)
    hm_ref[...] = (jnp.dot(xb, wc1a_ref[...], preferred_element_type=jnp.float32)
                   + jnp.dot(xa, wc1b_ref[...], preferred_element_type=jnp.float32))


def _front(x, W_lin1, b_lin1, W_lin2, b_lin2, W_conv1):
    W2p = jnp.zeros((H, H), jnp.float32).at[:AA, :AA].set(W_lin2)
    b2p = jnp.zeros((1, H), jnp.float32).at[0, :AA].set(b_lin2)
    Wc1a = jnp.zeros((H, H), jnp.float32).at[:AA, :].set(W_conv1[:AA])
    W1p = jnp.zeros((H, H), jnp.float32).at[AA:, :].set(W_lin1)
    b1p = jnp.broadcast_to(b_lin1[None, :], (1, H))
    Wc1b = W_conv1[AA:]
    bm = 512
    return pl.pallas_call(
        _front_body,
        grid=(NP // bm,),
        in_specs=[pl.BlockSpec((bm, H), lambda i: (i, 0))] +
                 [pl.BlockSpec((H, H), lambda i: (0, 0)),
                  pl.BlockSpec((1, H), lambda i: (0, 0)),
                  pl.BlockSpec((H, H), lambda i: (0, 0)),
                  pl.BlockSpec((H, H), lambda i: (0, 0)),
                  pl.BlockSpec((1, H), lambda i: (0, 0)),
                  pl.BlockSpec((H, H), lambda i: (0, 0))],
        out_specs=pl.BlockSpec((bm, H), lambda i: (i, 0)),
        out_shape=jax.ShapeDtypeStruct((NP, H), jnp.float32),
    )(x, W2p, b2p, Wc1a, W1p, b1p, Wc1b)


def _matmul_body(x_ref, w_ref, o_ref):
    o_ref[...] = jnp.dot(x_ref[...], w_ref[...],
                         preferred_element_type=jnp.float32)


def _matmul(x, w):
    bm = 512
    return pl.pallas_call(
        _matmul_body,
        grid=(NP // bm,),
        in_specs=[pl.BlockSpec((bm, H), lambda i: (i, 0)),
                  pl.BlockSpec((H, H), lambda i: (0, 0))],
        out_specs=pl.BlockSpec((bm, H), lambda i: (i, 0)),
        out_shape=jax.ShapeDtypeStruct((NP, H), jnp.float32),
    )(x, w)


def _gcn_combine_body(a0_ref, a1_ref, hm_ref, dinv_ref, b_ref, m_ref, o_ref):
    dinv = dinv_ref[...]
    o_ref[...] = m_ref[...] * jnp.maximum(
        a0_ref[...] + a1_ref[...] + dinv * dinv * hm_ref[...] + b_ref[...], 0.0)


def _gcn_combine(a0, a1, hm, dinv_col, b, mask_col):
    bm = 512
    return pl.pallas_call(
        _gcn_combine_body,
        grid=(NP // bm,),
        in_specs=[pl.BlockSpec((bm, H), lambda i: (i, 0)),
                  pl.BlockSpec((bm, H), lambda i: (i, 0)),
                  pl.BlockSpec((bm, H), lambda i: (i, 0)),
                  pl.BlockSpec((bm, 1), lambda i: (i, 0)),
                  pl.BlockSpec((1, H), lambda i: (0, 0)),
                  pl.BlockSpec((bm, 1), lambda i: (i, 0))],
        out_specs=pl.BlockSpec((bm, H), lambda i: (i, 0)),
        out_shape=jax.ShapeDtypeStruct((NP, H), jnp.float32),
    )(a0, a1, hm, dinv_col, b[None, :], mask_col)


def _deg_to_dinv_body(d0_ref, d1_ref, gdinv_ref, pdinv_ref):
    deg = d0_ref[...] + d1_ref[...]
    gdinv_ref[...] = jax.lax.rsqrt(deg + 1.0)
    pdinv_ref[...] = jnp.where(deg > 0, 1.0 / jnp.maximum(deg, 1e-12), 0.0)


def _deg_to_dinv(d0, d1):
    bm = 512
    return pl.pallas_call(
        _deg_to_dinv_body,
        grid=(NP // bm,),
        in_specs=[pl.BlockSpec((bm, 1), lambda i: (i, 0)),
                  pl.BlockSpec((bm, 1), lambda i: (i, 0))],
        out_specs=[pl.BlockSpec((bm, 1), lambda i: (i, 0)),
                   pl.BlockSpec((bm, 1), lambda i: (i, 0))],
        out_shape=[jax.ShapeDtypeStruct((NP, 1), jnp.float32),
                   jax.ShapeDtypeStruct((NP, 1), jnp.float32)],
    )(d0, d1)


def _den_to_dinv_body(d0_ref, d1_ref, gdinv_ref, pdinv_ref):
    deg = ((d0_ref[...] + d1_ref[...]) > 1e-16).astype(jnp.float32)
    gdinv_ref[...] = jax.lax.rsqrt(deg + 1.0)
    pdinv_ref[...] = deg


def _den_to_dinv(d0, d1):
    bm = 512
    return pl.pallas_call(
        _den_to_dinv_body,
        grid=(NP // bm,),
        in_specs=[pl.BlockSpec((bm, 1), lambda i: (i, 0)),
                  pl.BlockSpec((bm, 1), lambda i: (i, 0))],
        out_specs=[pl.BlockSpec((bm, 1), lambda i: (i, 0)),
                   pl.BlockSpec((bm, 1), lambda i: (i, 0))],
        out_shape=[jax.ShapeDtypeStruct((NP, 1), jnp.float32),
                   jax.ShapeDtypeStruct((NP, 1), jnp.float32)],
    )(d0, d1)


def _score_body(x_ref, a0_ref, a1_ref, s_ref):
    s_ref[...] = jnp.sum(jnp.abs(x_ref[...] - a0_ref[...] - a1_ref[...]),
                         axis=1, keepdims=True)


def _score(x, a0, a1):
    bm = 512
    return pl.pallas_call(
        _score_body,
        grid=(NP // bm,),
        in_specs=[pl.BlockSpec((bm, H), lambda i: (i, 0)),
                  pl.BlockSpec((bm, H), lambda i: (i, 0)),
                  pl.BlockSpec((bm, H), lambda i: (i, 0))],
        out_specs=pl.BlockSpec((bm, 1), lambda i: (i, 0)),
        out_shape=jax.ShapeDtypeStruct((NP, 1), jnp.float32),
    )(x, a0, a1)


def _topk_body(k, s_ref, v_ref, keep_ref, scale_ref, amax_ref):
    rows = s_ref.shape[0]
    score = s_ref[...]
    valid = v_ref[...] > 0.5
    ridx = jax.lax.broadcasted_iota(jnp.int32, (rows, H), 0)
    cidx = jax.lax.broadcasted_iota(jnp.int32, (rows, H), 1)
    flat = ridx * H + cidx
    bits = jax.lax.bitcast_convert_type(score, jnp.uint32)
    keys = jnp.where(bits >> 31 != 0, ~bits, bits | jnp.uint32(0x80000000))
    keys = jnp.where(valid, keys, jnp.uint32(0))
    sgn = jnp.uint32(0x80000000)
    ik = jax.lax.bitcast_convert_type(keys ^ sgn, jnp.int32)

    def body(i, carry):
        prefix, kk = carry
        bit = jnp.uint32(1) << (31 - i)
        cand = prefix | bit
        mask = ~(bit - jnp.uint32(1))
        icand = jax.lax.bitcast_convert_type(cand ^ sgn, jnp.int32)
        imasked = jax.lax.bitcast_convert_type((keys & mask) ^ sgn, jnp.int32)
        cnt = jnp.sum((imasked >= icand).astype(jnp.int32))
        take = cnt >= kk
        return (jnp.where(take, cand, prefix),
                jnp.where(take, kk, kk - cnt))

    thr, _ = jax.lax.fori_loop(0, 32, body, (jnp.uint32(0), jnp.int32(k)))
    ithr = jax.lax.bitcast_convert_type(thr ^ sgn, jnp.int32)
    gt = ik > ithr
    eq = valid & (ik == ithr)
    n_gt = jnp.sum(gt.astype(jnp.int32))

    ut = (jax.lax.broadcasted_iota(jnp.int32, (H, H), 0)
          <= jax.lax.broadcasted_iota(jnp.int32, (H, H), 1)).astype(jnp.float32)
    lt_strict = (jax.lax.broadcasted_iota(jnp.int32, (rows, rows), 1)
                 < jax.lax.broadcasted_iota(jnp.int32, (rows, rows), 0)
                 ).astype(jnp.float32)

    def cumsum2d(m):
        mf = m.astype(jnp.float32)
        within = jnp.dot(mf, ut, preferred_element_type=jnp.float32)
        row_tot = jnp.sum(mf, axis=1, keepdims=True)
        excl = jnp.dot(lt_strict, jnp.broadcast_to(row_tot, (rows, H)),
                       preferred_element_type=jnp.float32)
        return (within + excl).astype(jnp.int32)

    eq_rank = cumsum2d(eq) - 1
    keep = gt | (eq & (eq_rank < (k - n_gt)))
    maxk = jnp.max(ik)
    amax = jnp.min(jnp.where(ik == maxk, flat, jnp.int32(2**30)))
    keep_ref[...] = keep.astype(jnp.float32)
    scale_ref[...] = jnp.where(keep, jnp.tanh(score), 0.0)
    amax_ref[...] = jnp.full((1, H), amax, jnp.int32)


def _topk(score_col, valid_col, k):
    rows = NP // H
    s_rs = jnp.reshape(score_col[:, 0], (rows, H))
    v_rs = jnp.reshape(valid_col[:, 0], (rows, H))
    keep_rs, scale_rs, amax_o = pl.pallas_call(
        partial(_topk_body, k),
        in_specs=[pl.BlockSpec((rows, H), lambda: (0, 0)),
                  pl.BlockSpec((rows, H), lambda: (0, 0))],
        out_specs=[pl.BlockSpec((rows, H), lambda: (0, 0)),
                   pl.BlockSpec((rows, H), lambda: (0, 0)),
                   pl.BlockSpec((1, H), lambda: (0, 0))],
        out_shape=[jax.ShapeDtypeStruct((rows, H), jnp.float32),
                   jax.ShapeDtypeStruct((rows, H), jnp.float32),
                   jax.ShapeDtypeStruct((1, H), jnp.int32)],
    )(s_rs, v_rs)
    keep = jnp.reshape(keep_rs, (NP, 1))
    scale = jnp.reshape(scale_rs, (NP, 1))
    return keep, scale, amax_o[0, :16]


def _scale_rows_body(x_ref, t_ref, o_ref):
    o_ref[...] = x_ref[...] * t_ref[...]


def _scale_rows(x, t_col):
    bm = 512
    return pl.pallas_call(
        _scale_rows_body,
        grid=(NP // bm,),
        in_specs=[pl.BlockSpec((bm, H), lambda i: (i, 0)),
                  pl.BlockSpec((bm, 1), lambda i: (i, 0))],
        out_specs=pl.BlockSpec((bm, H), lambda i: (i, 0)),
        out_shape=jax.ShapeDtypeStruct((NP, H), jnp.float32),
    )(x, t_col)


def _att_readout_body(xk_ref, atta_ref, attb_ref, ca_ref, cb_ref, mx_ref,
                      sm_ref):
    xk = xk_ref[...]
    ca_ref[...] = jnp.sum(xk * atta_ref[...], axis=1, keepdims=True)
    cb_ref[...] = jnp.sum(xk * attb_ref[...], axis=1, keepdims=True)
    i = pl.program_id(0)
    bmax = jnp.max(xk, axis=0, keepdims=True)
    bsum = jnp.sum(xk, axis=0, keepdims=True)

    @pl.when(i == 0)
    def _():
        mx_ref[...] = bmax
        sm_ref[...] = bsum

    @pl.when(i != 0)
    def _():
        mx_ref[...] = jnp.maximum(mx_ref[...], bmax)
        sm_ref[...] = sm_ref[...] + bsum


def _att_readout(xk, att):
    bm = 512
    atta = att[None, :H]
    attb = att[None, H:]
    return pl.pallas_call(
        _att_readout_body,
        grid=(NP // bm,),
        in_specs=[pl.BlockSpec((bm, H), lambda i: (i, 0)),
                  pl.BlockSpec((1, H), lambda i: (0, 0)),
                  pl.BlockSpec((1, H), lambda i: (0, 0))],
        out_specs=[pl.BlockSpec((bm, 1), lambda i: (i, 0)),
                   pl.BlockSpec((bm, 1), lambda i: (i, 0)),
                   pl.BlockSpec((1, H), lambda i: (0, 0)),
                   pl.BlockSpec((1, H), lambda i: (0, 0))],
        out_shape=[jax.ShapeDtypeStruct((NP, 1), jnp.float32),
                   jax.ShapeDtypeStruct((NP, 1), jnp.float32),
                   jax.ShapeDtypeStruct((1, H), jnp.float32),
                   jax.ShapeDtypeStruct((1, H), jnp.float32)],
    )(xk, atta, attb)


def _final_combine_body(a0_ref, a1_ref, hm_ref, dinv_ref, b_ref, m_ref,
                        zs_ref, an_ref, mx_ref, sm_ref):
    dinv = dinv_ref[...]
    i = pl.program_id(0)
    bm = hm_ref.shape[0]
    zw = jnp.sum(zs_ref[...])
    a1s = an_ref[0, 0]
    rowid = i * bm + jax.lax.broadcasted_iota(jnp.int32, (bm, H), 0)
    factor = jnp.where(rowid == a1s, 1.0 + zw, 1.0)
    h = m_ref[...] * jnp.maximum(
        a0_ref[...] + a1_ref[...] + dinv * dinv * hm_ref[...] * factor
        + b_ref[...], 0.0)
    bmax = jnp.max(h, axis=0, keepdims=True)
    bsum = jnp.sum(h, axis=0, keepdims=True)

    @pl.when(i == 0)
    def _():
        mx_ref[...] = bmax
        sm_ref[...] = bsum

    @pl.when(i != 0)
    def _():
        mx_ref[...] = jnp.maximum(mx_ref[...], bmax)
        sm_ref[...] = sm_ref[...] + bsum


def _final_combine(a0, a1, hm, dinv_col, b, mask_col, zs_rs, a1_bc):
    bm = 512
    return pl.pallas_call(
        _final_combine_body,
        grid=(NP // bm,),
        in_specs=[pl.BlockSpec((bm, H), lambda i: (i, 0)),
                  pl.BlockSpec((bm, H), lambda i: (i, 0)),
                  pl.BlockSpec((bm, H), lambda i: (i, 0)),
                  pl.BlockSpec((bm, 1), lambda i: (i, 0)),
                  pl.BlockSpec((1, H), lambda i: (0, 0)),
                  pl.BlockSpec((bm, 1), lambda i: (i, 0)),
                  pl.BlockSpec((4, H), lambda i: (0, 0)),
                  pl.BlockSpec((1, H), lambda i: (0, 0))],
        out_specs=[pl.BlockSpec((1, H), lambda i: (0, 0)),
                   pl.BlockSpec((1, H), lambda i: (0, 0))],
        out_shape=[jax.ShapeDtypeStruct((1, H), jnp.float32),
                   jax.ShapeDtypeStruct((1, H), jnp.float32)],
    )(a0, a1, hm, dinv_col, b[None, :], mask_col, zs_rs, a1_bc)


def _head_body(k1_inv, k2_inv, k3_inv,
               mx1_ref, sm1_ref, mx2_ref, sm2_ref, mx3_ref, sm3_ref,
               wf1a_ref, wf1b_ref, bf1_ref, wf2_ref, bf2_ref, wf3_ref,
               bf3_ref, o_ref):
    ra = (jnp.maximum(mx1_ref[...], 0.0) + jnp.maximum(mx2_ref[...], 0.0)
          + jnp.maximum(mx3_ref[...], 0.0))
    rb = (jnp.maximum(sm1_ref[...] * k1_inv, 0.0)
          + jnp.maximum(sm2_ref[...] * k2_inv, 0.0)
          + jnp.maximum(sm3_ref[...] * k3_inv, 0.0))
    o = jnp.maximum(
        jnp.dot(ra, wf1a_ref[...], preferred_element_type=jnp.float32)
        + jnp.dot(rb, wf1b_ref[...], preferred_element_type=jnp.float32)
        + bf1_ref[...], 0.0)
    o = jnp.maximum(
        jnp.dot(o, wf2_ref[...], preferred_element_type=jnp.float32)
        + bf2_ref[...], 0.0)
    lg = jnp.dot(o, wf3_ref[...], preferred_element_type=jnp.float32) \
        + bf3_ref[...]
    lane = jax.lax.broadcasted_iota(jnp.int32, (1, H), 1)
    lvalid = lane < 2
    m = jnp.max(jnp.where(lvalid, lg, -jnp.inf))
    s = jnp.sum(jnp.where(lvalid, jnp.exp(lg - m), 0.0))
    o_ref[...] = lg - m - jnp.log(s)


def _head(mx1, sm1, mx2, sm2, mx3, sm3, W_fc1, b_fc1, W_fc2, b_fc2,
          W_fc3, b_fc3):
    wf1a = W_fc1[:H]
    wf1b = W_fc1[H:]
    wf2 = jnp.zeros((H, H), jnp.float32).at[:, :H // 2].set(W_fc2)
    bf2 = jnp.zeros((1, H), jnp.float32).at[0, :H // 2].set(b_fc2)
    wf3 = jnp.zeros((H, H), jnp.float32).at[:H // 2, :2].set(W_fc3)
    bf3 = jnp.zeros((1, H), jnp.float32).at[0, :2].set(b_fc3)
    out = pl.pallas_call(
        partial(_head_body, 1.0 / K1, 1.0 / K2, 1.0 / K2),
        in_specs=[pl.BlockSpec((1, H), lambda: (0, 0))] * 6 +
                 [pl.BlockSpec((H, H), lambda: (0, 0)),
                  pl.BlockSpec((H, H), lambda: (0, 0)),
                  pl.BlockSpec((1, H), lambda: (0, 0)),
                  pl.BlockSpec((H, H), lambda: (0, 0)),
                  pl.BlockSpec((1, H), lambda: (0, 0)),
                  pl.BlockSpec((H, H), lambda: (0, 0)),
                  pl.BlockSpec((1, H), lambda: (0, 0))],
        out_specs=pl.BlockSpec((1, H), lambda: (0, 0)),
        out_shape=jax.ShapeDtypeStruct((1, H), jnp.float32),
    )(mx1, sm1, mx2, sm2, mx3, sm3, wf1a, wf1b, b_fc1[None, :], wf2, bf2,
      wf3, bf3)
    return out[:, :2]


def kernel(x, edge_index, batch, W_lin1, b_lin1, W_lin2, b_lin2, W_conv1,
           b_conv1, W_conv2, b_conv2, W_conv3, b_conv3, att1, att2, W_fc1,
           b_fc1, W_fc2, b_fc2, W_fc3, b_fc3):
    xp = jnp.zeros((NP, H), jnp.float32).at[:N0].set(x)
    src = edge_index[0].astype(jnp.int32)
    dst = edge_index[1].astype(jnp.int32)
    pad = jnp.full((EP - E,), DEAD, jnp.int32)
    src2 = jnp.reshape(jnp.concatenate([src, pad]), (ER, 128))
    dst2 = jnp.reshape(jnp.concatenate([dst, pad]), (ER, 128))
    w0 = jnp.reshape(
        jnp.concatenate([jnp.ones((E,), jnp.float32),
                         jnp.zeros((EP - E,), jnp.float32)]), (ER, 128))
    keep0 = (jnp.arange(NP, dtype=jnp.int32) < N0).astype(jnp.float32)[:, None]

    hm1 = _front(xp, W_lin1, b_lin1, W_lin2, b_lin2, W_conv1)
    degp = _sc_deg(dst2)
    gdinv1, pdinv1 = _deg_to_dinv(degp[0][:, None], degp[1][:, None])
    agg = _sc_agg(hm1, src2, dst2, w0, gdinv1[:, 0], 'both')
    h1 = _gcn_combine(agg[0], agg[1], hm1, gdinv1, b_conv1, keep0)

    aggp = _sc_agg(h1, src2, dst2, w0, pdinv1[:, 0], 'dst')
    score1 = _score(h1, aggp[0], aggp[1])
    keep1, scale1, a1v = _topk(score1, keep0, K1)
    xz1 = _scale_rows(h1, scale1)
    ca1, cb1, mx1, sm1 = _att_readout(xz1, att1)
    ee1, den1 = _sc_attn1a(src2, dst2, ca1[:, 0], cb1[:, 0], keep1[:, 0])
    w1 = _sc_attn_norm(ee1, dst2, den1)
    gdinv2, pdinv2 = _den_to_dinv(den1[0][:, None], den1[1][:, None])

    hm2 = _matmul(xz1, W_conv2)
    agg = _sc_agg(hm2, src2, dst2, w1, gdinv2[:, 0], 'both')
    h2 = _gcn_combine(agg[0], agg[1], hm2, gdinv2, b_conv2, keep1)

    aggp = _sc_agg(h2, src2, dst2, w1, pdinv2[:, 0], 'dst')
    score2 = _score(h2, aggp[0], aggp[1])
    keep2, scale2, _ = _topk(score2, keep1, K2)
    xz2 = _scale_rows(h2, scale2)
    ca2, cb2, mx2, sm2 = _att_readout(xz2, att2)
    sh2, dh2, ee2, den2 = _sc_attn2a(src2, dst2, w1, ca2[:, 0], cb2[:, 0],
                                     keep1[:, 0], keep2[:, 0], a1v)
    w2 = _sc_attn_norm(ee2, dh2, den2)
    gdinv3, _ = _den_to_dinv(den2[0][:, None], den2[1][:, None])

    hm3 = _matmul(xz2, W_conv3)
    agg, zs = _sc_agg(hm3, sh2, dh2, w2, gdinv3[:, 0], 'both', a1v)
    zs_rs = jnp.reshape(zs, (4, H))
    a1_bc = jnp.broadcast_to(a1v[:1][None, :], (1, H))
    mx3, sm3 = _final_combine(agg[0], agg[1], hm3, gdinv3, b_conv3, keep2,
                              zs_rs, a1_bc)

    return _head(mx1, sm1, mx2, sm2, mx3, sm3, W_fc1, b_fc1, W_fc2, b_fc2,
                 W_fc3, b_fc3)
